# Initial kernel scaffold; baseline (speedup 1.0000x reference)
import jax
import jax.numpy as jnp
from jax.experimental import pallas as pl


def kernel(user_emb, item_emb, behavior_weights, batch_data, edges_view, edges_cart, edges_buy):
    raise NotImplementedError("write your pallas kernel here")


# speedup vs baseline: 16.2283x; 16.2283x over previous
import functools

import jax
import jax.numpy as jnp
from jax import lax
from jax.experimental import pallas as pl
from jax.experimental.pallas import tpu as pltpu
from jax.experimental.pallas import tpu_sc as plsc

N_USERS = 25000
N_ITEMS = 25000
D = 64
E = 800000
N_BEH = 3
B = 4096
REG_WEIGHT = 1e-3
NU = N_USERS + 1
N = NU + (N_ITEMS + 1)
HD = D // 2

CH = 128
NCHUNK = E // CH
NSUB = 16
NCORE = 2

NPAD = 51200
TROWS = NPAD // NSUB
ZR = CH
DPAD = 150016
DROWS = DPAD // NSUB

@functools.lru_cache(maxsize=None)
def _mesh():
    return plsc.VectorSubcoreMesh(core_axis_name="c", subcore_axis_name="s",
                                  num_cores=NCORE, num_subcores=NSUB)


_SC_PARAMS = pltpu.CompilerParams(use_tc_tiling_on_sc=False)


def _build_idx(dst_ref, src_ref, offset):
    for t in range(CH // 16):
        sl = pl.ds(t * 16, 16)
        dst_ref[sl] = src_ref[sl] + offset


def _deg_body(eu0, ei0, eu1, ei1, eu2, ei2, zeros, out,
              u_buf, i_buf, gu_buf, gi_buf, ones, vbuf, acc, sem):
    c = lax.axis_index("c")
    s = lax.axis_index("s")
    wid = s * NCORE + c

    for t in range(CH // 16):
        ones[pl.ds(t * 16, 16)] = jnp.full((16,), 1.0, jnp.float32)

    lo = s * DROWS
    pltpu.sync_copy(zeros, vbuf)
    pltpu.sync_copy(vbuf, acc.at[pl.ds(lo, DROWS)])

    plsc.subcore_barrier()

    for b, (eu, ei) in enumerate(((eu0, ei0), (eu1, ei1), (eu2, ei2))):
        bN = b * N

        def chunk(k, _, eu=eu, ei=ei, bN=bN):
            off = (wid + 32 * k) * CH
            pltpu.sync_copy(eu.at[pl.ds(off, CH)], u_buf)
            pltpu.sync_copy(ei.at[pl.ds(off, CH)], i_buf)
            _build_idx(gu_buf, u_buf, bN)
            _build_idx(gi_buf, i_buf, bN + NU)
            pltpu.sync_copy(ones, acc.at[gu_buf], add=True)
            pltpu.sync_copy(ones, acc.at[gi_buf], add=True)
            return 0

        nk = (NCHUNK - wid + 31) // 32
        lax.fori_loop(0, nk, chunk, 0)

    plsc.subcore_barrier()

    pltpu.sync_copy(acc.at[pl.ds(lo, DROWS)], vbuf)
    pltpu.sync_copy(vbuf, out.at[pl.ds(c * DPAD + lo, DROWS)])


@functools.lru_cache(maxsize=None)
def _deg_kernel():
    return functools.partial(
        pl.kernel,
        out_type=jax.ShapeDtypeStruct((NCORE * DPAD,), jnp.float32),
        mesh=_mesh(),
        compiler_params=_SC_PARAMS,
        scratch_types=[
            pltpu.VMEM((CH,), jnp.int32),
            pltpu.VMEM((CH,), jnp.int32),
            pltpu.VMEM((CH,), jnp.int32),
            pltpu.VMEM((CH,), jnp.int32),
            pltpu.VMEM((CH,), jnp.float32),
            pltpu.VMEM((DROWS,), jnp.float32),
            pltpu.VMEM_SHARED((DPAD,), jnp.float32),
            pltpu.SemaphoreType.DMA,
        ],
    )(_deg_body)


def _make_prop_body(n_edges):
    def body(*refs):
        table = refs[0]
        pairs = [(refs[1 + 2 * j], refs[2 + 2 * j]) for j in range(n_edges)]
        zeros = refs[1 + 2 * n_edges]
        out = refs[2 + 2 * n_edges]
        (u_buf, i_buf, g_buf, g2_buf, s_buf, rows, rows2, acc, sem) = \
            refs[3 + 2 * n_edges:]

        c = lax.axis_index("c")
        s = lax.axis_index("s")
        cN = c * N

        lo = s * TROWS
        pltpu.sync_copy(zeros, rows)

        def zchunk(k, _):
            pltpu.sync_copy(rows, acc.at[pl.ds(lo + k * ZR, ZR)])
            return 0

        lax.fori_loop(0, TROWS // ZR, zchunk, 0)

        plsc.subcore_barrier()

        for eu, ei in pairs:
            def chunk(k, _, eu=eu, ei=ei):
                off = (s + NSUB * k) * CH
                pltpu.sync_copy(eu.at[pl.ds(off, CH)], u_buf)
                pltpu.sync_copy(ei.at[pl.ds(off, CH)], i_buf)
                _build_idx(g_buf, u_buf, cN)
                _build_idx(s_buf, i_buf, NU)
                _build_idx(g2_buf, i_buf, cN + NU)
                pltpu.async_copy(table.at[g_buf], rows, sem).wait()
                pltpu.sync_copy(rows, acc.at[s_buf], add=True)
                pltpu.async_copy(table.at[g2_buf], rows2, sem).wait()
                pltpu.sync_copy(rows2, acc.at[u_buf], add=True)
                return 0

            nk = (NCHUNK - s + NSUB - 1) // NSUB
            lax.fori_loop(0, nk, chunk, 0)

        plsc.subcore_barrier()

        def ochunk(k, _):
            pltpu.sync_copy(acc.at[pl.ds(lo + k * ZR, ZR)], rows)
            pltpu.sync_copy(rows, out.at[pl.ds(c * NPAD + lo + k * ZR, ZR)])
            return 0

        lax.fori_loop(0, TROWS // ZR, ochunk, 0)

    return body


@functools.lru_cache(maxsize=None)
def _make_prop_kernel(n_edges):
    body = _make_prop_body(n_edges)
    return functools.partial(
        pl.kernel,
        out_type=jax.ShapeDtypeStruct((NCORE * NPAD, HD), jnp.float32),
        mesh=_mesh(),
        compiler_params=_SC_PARAMS,
        scratch_types=[
            pltpu.VMEM((CH,), jnp.int32),
            pltpu.VMEM((CH,), jnp.int32),
            pltpu.VMEM((CH,), jnp.int32),
            pltpu.VMEM((CH,), jnp.int32),
            pltpu.VMEM((CH,), jnp.int32),
            pltpu.VMEM((CH, HD), jnp.float32),
            pltpu.VMEM((CH, HD), jnp.float32),
            pltpu.VMEM_SHARED((NPAD, HD), jnp.float32),
            pltpu.SemaphoreType.DMA,
        ],
    )(body)


def _gath_body(table, pairT, out, jbuf, gbuf, rows, sem):
    c = lax.axis_index("c")
    s = lax.axis_index("s")
    cN = c * N
    per_tile = B // NSUB
    for k in range(3):
        for q in range(per_tile // CH):
            off = s * per_tile + q * CH
            pltpu.sync_copy(pairT.at[k, pl.ds(off, CH)], jbuf)
            _build_idx(gbuf, jbuf, cN if k == 0 else cN + NU)
            pltpu.async_copy(table.at[gbuf], rows, sem).wait()
            base = c * (3 * B) + k * B + off
            pltpu.sync_copy(rows, out.at[pl.ds(base, CH)])


@functools.lru_cache(maxsize=None)
def _gath_kernel():
    return functools.partial(
        pl.kernel,
        out_type=jax.ShapeDtypeStruct((NCORE * 3 * B, HD), jnp.float32),
        mesh=_mesh(),
        compiler_params=_SC_PARAMS,
        scratch_types=[
            pltpu.VMEM((CH,), jnp.int32),
            pltpu.VMEM((CH,), jnp.int32),
            pltpu.VMEM((CH, HD), jnp.float32),
            pltpu.SemaphoreType.DMA,
        ],
    )(_gath_body)


NB = 2048
GN = -(-N // NB)


def _dinv_of(d):
    return jnp.where(d > 0.0, lax.rsqrt(jnp.maximum(d, 1.0)), 0.0)


def _prep_body(x0_ref, degp_ref, dinv_ref, z0_ref, y_ref):
    deg = degp_ref[...]
    degs = deg[0] + deg[1]
    degh = degs[0] + degs[1] + degs[2]
    dh = _dinv_of(degh)
    dinv_ref[0] = dh
    for b in range(3):
        dinv_ref[1 + b] = _dinv_of(degs[b])
    x = x0_ref[...]
    z0_ref[0] = x[:, :HD]
    z0_ref[1] = x[:, HD:]
    y_ref[0] = x[:, :HD] * dh
    y_ref[1] = x[:, HD:] * dh


def _tc_prep(x0, degp):
    return pl.pallas_call(
        _prep_body,
        grid=(GN,),
        in_specs=[
            pl.BlockSpec((NB, D), lambda j: (j, 0)),
            pl.BlockSpec((2, 3, NB, 1), lambda j: (0, 0, j, 0)),
        ],
        out_specs=[
            pl.BlockSpec((4, NB, 1), lambda j: (0, j, 0)),
            pl.BlockSpec((2, NB, HD), lambda j: (0, j, 0)),
            pl.BlockSpec((2, NB, HD), lambda j: (0, j, 0)),
        ],
        out_shape=[
            jax.ShapeDtypeStruct((4, N, 1), jnp.float32),
            jax.ShapeDtypeStruct((2, N, HD), jnp.float32),
            jax.ShapeDtypeStruct((2, N, HD), jnp.float32),
        ],
    )(x0, degp)


def _hdgfin_body(z0_ref, s_ref, dinv_ref, total_ref, y0_ref):
    dh = dinv_ref[0][None]
    d0 = dinv_ref[1][None]
    tot = 0.5 * (z0_ref[...] + s_ref[...] * dh)
    total_ref[...] = tot
    y0_ref[...] = tot * d0


def _tc_hdgfin(z0, s, dinv):
    return pl.pallas_call(
        _hdgfin_body,
        grid=(GN,),
        in_specs=[
            pl.BlockSpec((2, NB, HD), lambda j: (0, j, 0)),
            pl.BlockSpec((2, NB, HD), lambda j: (0, j, 0)),
            pl.BlockSpec((4, NB, 1), lambda j: (0, j, 0)),
        ],
        out_specs=[
            pl.BlockSpec((2, NB, HD), lambda j: (0, j, 0)),
            pl.BlockSpec((2, NB, HD), lambda j: (0, j, 0)),
        ],
        out_shape=[
            jax.ShapeDtypeStruct((2, N, HD), jnp.float32),
            jax.ShapeDtypeStruct((2, N, HD), jnp.float32),
        ],
    )(z0, s, dinv)


def _make_midscale_body(b):
    def body(s1_ref, dinv_ref, y1_ref):
        db = dinv_ref[1 + b][None]
        y1_ref[...] = s1_ref[...] * (db * db)
    return body


def _tc_midscale(s1, dinv, b):
    return pl.pallas_call(
        _make_midscale_body(b),
        grid=(GN,),
        in_specs=[
            pl.BlockSpec((2, NB, HD), lambda j: (0, j, 0)),
            pl.BlockSpec((4, NB, 1), lambda j: (0, j, 0)),
        ],
        out_specs=pl.BlockSpec((2, NB, HD), lambda j: (0, j, 0)),
        out_shape=jax.ShapeDtypeStruct((2, N, HD), jnp.float32),
    )(s1, dinv)


def _make_combine_body(b, last):
    def body(*refs):
        if b == 0:
            total_ref, s1_ref, s2_ref, dinv_ref, bw_ref = refs[:5]
            outs = refs[5:]
            acc_prev = None
        else:
            total_ref, s1_ref, s2_ref, dinv_ref, bw_ref, acc_ref = refs[:6]
            outs = refs[6:]
            acc_prev = acc_ref[...]
        if last:
            acc_out_ref, = outs
        else:
            total_out_ref, acc_out_ref, ynext_ref = outs

        db = dinv_ref[1 + b][None]
        total = total_ref[...]
        h1 = s1_ref[...] * db
        h2 = s2_ref[...] * db
        layer = (total + h1 + h2) * (1.0 / 3.0)
        ss = jnp.sum(layer * layer, axis=(0, 2))
        scale = (1.0 / jnp.maximum(jnp.sqrt(ss), 1e-12))[None, :, None]
        tot2 = total + layer * scale
        sw = 1.0 / (1.0 + jnp.exp(-bw_ref[b]))
        acc2 = sw * tot2 if acc_prev is None else acc_prev + sw * tot2
        acc_out_ref[...] = acc2
        if not last:
            total_out_ref[...] = tot2
            ynext_ref[...] = tot2 * dinv_ref[2 + b][None]
    return body


def _tc_combine(total, s1, s2, dinv, bw, acc_prev, b):
    last = b == 2
    blk = pl.BlockSpec((2, NB, HD), lambda j: (0, j, 0))
    in_specs = [
        blk, blk, blk,
        pl.BlockSpec((4, NB, 1), lambda j: (0, j, 0)),
        pl.BlockSpec(memory_space=pltpu.SMEM),
    ]
    args = [total, s1, s2, dinv, bw]
    if b > 0:
        in_specs.append(blk)
        args.append(acc_prev)
    if last:
        out_specs = [blk]
        out_shape = [jax.ShapeDtypeStruct((2, N, HD), jnp.float32)]
    else:
        out_specs = [blk, blk, blk]
        out_shape = [jax.ShapeDtypeStruct((2, N, HD), jnp.float32)] * 3
    res = pl.pallas_call(
        _make_combine_body(b, last),
        grid=(GN,),
        in_specs=in_specs,
        out_specs=out_specs,
        out_shape=out_shape,
    )(*args)
    if last:
        return None, res[0], None
    return res[0], res[1], res[2]


BJ = 1024
GJ = B // BJ


def _loss_body(g_ref, p_ref, s1_ref, s2_ref):
    j = pl.program_id(0)
    g = g_ref[...]
    u = g[:, 0]
    i1 = g[:, 1]
    i2 = g[:, 2]
    sp = jnp.sum(u * i1, axis=(0, 2))
    sn = jnp.sum(u * i2, axis=(0, 2))
    z = sp - sn
    vals = jnp.where(z > 0.0, -jnp.log1p(jnp.exp(-z)), z - jnp.log1p(jnp.exp(z)))
    m = jnp.any(p_ref[...] != 0, axis=0).astype(jnp.float32)

    @pl.when(j == 0)
    def _():
        s1_ref[...] = jnp.zeros((1, 1), jnp.float32)
        s2_ref[...] = jnp.zeros((1, 1), jnp.float32)

    s1_ref[...] += jnp.sum(vals * m).reshape(1, 1)
    s2_ref[...] += jnp.sum(m).reshape(1, 1)


def _tc_loss(gath, pairT):
    return pl.pallas_call(
        _loss_body,
        grid=(GJ,),
        in_specs=[
            pl.BlockSpec((2, 3, BJ, HD), lambda j: (0, 0, j, 0)),
            pl.BlockSpec((3, BJ), lambda j: (0, j)),
        ],
        out_specs=[
            pl.BlockSpec((1, 1), lambda j: (0, 0)),
            pl.BlockSpec((1, 1), lambda j: (0, 0)),
        ],
        out_shape=[
            jax.ShapeDtypeStruct((1, 1), jnp.float32),
            jax.ShapeDtypeStruct((1, 1), jnp.float32),
        ],
    )(gath, pairT)


BU = 2048
GU = -(-NU // BU)


def _frob_body(u_ref, i_ref, su_ref, si_ref):
    j = pl.program_id(0)
    rows = lax.broadcasted_iota(jnp.int32, (BU, D), 0) + j * BU
    msk = (rows < NU).astype(jnp.float32)
    u = u_ref[...] * msk
    v = i_ref[...] * msk

    @pl.when(j == 0)
    def _():
        su_ref[...] = jnp.zeros((1, 1), jnp.float32)
        si_ref[...] = jnp.zeros((1, 1), jnp.float32)

    su_ref[...] += jnp.sum(u * u).reshape(1, 1)
    si_ref[...] += jnp.sum(v * v).reshape(1, 1)


def _tc_frob(user_emb, item_emb):
    return pl.pallas_call(
        _frob_body,
        grid=(GU,),
        in_specs=[
            pl.BlockSpec((BU, D), lambda j: (j, 0)),
            pl.BlockSpec((BU, D), lambda j: (j, 0)),
        ],
        out_specs=[
            pl.BlockSpec((1, 1), lambda j: (0, 0)),
            pl.BlockSpec((1, 1), lambda j: (0, 0)),
        ],
        out_shape=[
            jax.ShapeDtypeStruct((1, 1), jnp.float32),
            jax.ShapeDtypeStruct((1, 1), jnp.float32),
        ],
    )(user_emb, item_emb)


def kernel(user_emb, item_emb, behavior_weights, batch_data,
           edges_view, edges_cart, edges_buy):
    f32 = jnp.float32
    _prop1 = _make_prop_kernel(1)
    _prop3 = _make_prop_kernel(3)
    eu = [e[0] for e in (edges_view, edges_cart, edges_buy)]
    ei = [e[1] for e in (edges_view, edges_cart, edges_buy)]
    zeros_deg = jnp.zeros((DROWS,), f32)
    zeros_prop = jnp.zeros((ZR, HD), f32)
    x0 = jnp.concatenate([user_emb, item_emb], axis=0)

    degp = _deg_kernel()(eu[0], ei[0], eu[1], ei[1], eu[2], ei[2], zeros_deg)
    degp = degp.reshape(2, DPAD)[:, :3 * N].reshape(2, 3, N, 1)
    dinv, z0, y = _tc_prep(x0, degp)

    s = _prop3(y.reshape(2 * N, HD), eu[0], ei[0], eu[1], ei[1],
               eu[2], ei[2], zeros_prop)
    total, ynext = _tc_hdgfin(z0, s.reshape(2, NPAD, HD), dinv)

    acc = None
    for b in range(3):
        s1 = _prop1(ynext.reshape(2 * N, HD), eu[b], ei[b], zeros_prop)
        s1 = s1.reshape(2, NPAD, HD)
        y1 = _tc_midscale(s1, dinv, b)
        s2 = _prop1(y1.reshape(2 * N, HD), eu[b], ei[b], zeros_prop)
        s2 = s2.reshape(2, NPAD, HD)
        total, acc, ynext = _tc_combine(total, s1, s2, dinv,
                                        behavior_weights, acc, b)

    pairT = batch_data[:, -1, :3].T
    gath = _gath_kernel()(acc.reshape(2 * N, HD), pairT)
    s1_, s2_ = _tc_loss(gath.reshape(2, 3, B, HD), pairT)
    su, si = _tc_frob(user_emb, item_emb)

    bpr = -s1_[0, 0] / s2_[0, 0]
    emb = (jnp.sqrt(su[0, 0]) + jnp.sqrt(si[0, 0])) / (N_ITEMS + 1)
    return bpr + REG_WEIGHT * emb


# speedup vs baseline: 33.1616x; 2.0434x over previous
import functools

import jax
import jax.numpy as jnp
from jax import lax
from jax.experimental import pallas as pl
from jax.experimental.pallas import tpu as pltpu
from jax.experimental.pallas import tpu_sc as plsc

N_USERS = 25000
N_ITEMS = 25000
D = 64
E = 800000
N_BEH = 3
B = 4096
REG_WEIGHT = 1e-3
NU = N_USERS + 1
N = NU + (N_ITEMS + 1)
HD = D // 2

CH = 128
NCHUNK = E // CH
NSUB = 16
NCORE = 2

NPAD = 51200
TROWS = NPAD // NSUB
ZR = CH
BLK = 256
NJ = BLK // CH
NBLK = E // BLK
DPAD = 150016
DROWS = DPAD // NSUB

@functools.lru_cache(maxsize=None)
def _mesh():
    return plsc.VectorSubcoreMesh(core_axis_name="c", subcore_axis_name="s",
                                  num_cores=NCORE, num_subcores=NSUB)


_SC_PARAMS = pltpu.CompilerParams(use_tc_tiling_on_sc=False)


def _build_idx(dst_ref, src_ref, offset):
    for t in range(CH // 16):
        sl = pl.ds(t * 16, 16)
        dst_ref[sl] = src_ref[sl] + offset


def _deg_body(eu0, ei0, eu1, ei1, eu2, ei2, zeros, out,
              u_buf, i_buf, gu_buf, gi_buf, ones, vbuf, acc, sem):
    c = lax.axis_index("c")
    s = lax.axis_index("s")
    wid = s * NCORE + c

    for t in range(CH // 16):
        ones[pl.ds(t * 16, 16)] = jnp.full((16,), 1.0, jnp.float32)

    lo = s * DROWS
    pltpu.sync_copy(zeros, vbuf)
    pltpu.sync_copy(vbuf, acc.at[pl.ds(lo, DROWS)])

    plsc.subcore_barrier()

    for b, (eu, ei) in enumerate(((eu0, ei0), (eu1, ei1), (eu2, ei2))):
        bN = b * N

        def chunk(k, _, eu=eu, ei=ei, bN=bN):
            off = (wid + 32 * k) * CH
            pltpu.sync_copy(eu.at[pl.ds(off, CH)], u_buf)
            pltpu.sync_copy(ei.at[pl.ds(off, CH)], i_buf)
            _build_idx(gu_buf, u_buf, bN)
            _build_idx(gi_buf, i_buf, bN + NU)
            pltpu.sync_copy(ones, acc.at[gu_buf], add=True)
            pltpu.sync_copy(ones, acc.at[gi_buf], add=True)
            return 0

        nk = (NCHUNK - wid + 31) // 32
        lax.fori_loop(0, nk, chunk, 0)

    plsc.subcore_barrier()

    pltpu.sync_copy(acc.at[pl.ds(lo, DROWS)], vbuf)
    pltpu.sync_copy(vbuf, out.at[pl.ds(c * DPAD + lo, DROWS)])


@functools.lru_cache(maxsize=None)
def _deg_kernel():
    return functools.partial(
        pl.kernel,
        out_type=jax.ShapeDtypeStruct((NCORE * DPAD,), jnp.float32),
        mesh=_mesh(),
        compiler_params=_SC_PARAMS,
        scratch_types=[
            pltpu.VMEM((CH,), jnp.int32),
            pltpu.VMEM((CH,), jnp.int32),
            pltpu.VMEM((CH,), jnp.int32),
            pltpu.VMEM((CH,), jnp.int32),
            pltpu.VMEM((CH,), jnp.float32),
            pltpu.VMEM((DROWS,), jnp.float32),
            pltpu.VMEM_SHARED((DPAD,), jnp.float32),
            pltpu.SemaphoreType.DMA,
        ],
    )(_deg_body)


def _make_prop_body(n_edges):
    def body(*refs):
        table = refs[0]
        pairs = [(refs[1 + 2 * j], refs[2 + 2 * j]) for j in range(n_edges)]
        zeros = refs[1 + 2 * n_edges]
        out = refs[2 + 2 * n_edges]
        (eu_buf, ei_buf, gu2d, su2d, gi2d, si2d, rows_u, rows_i,
         acc, semg, sems, seme) = refs[3 + 2 * n_edges:]

        c = lax.axis_index("c")
        s = lax.axis_index("s")
        cN = c * N

        lo = s * TROWS
        pltpu.sync_copy(zeros, rows_u.at[pl.ds(0, ZR)])

        def zchunk(k, _):
            pltpu.sync_copy(rows_u.at[pl.ds(0, ZR)],
                            acc.at[pl.ds(lo + k * ZR, ZR)])
            return 0

        lax.fori_loop(0, TROWS // ZR, zchunk, 0)

        plsc.subcore_barrier()

        for eu, ei in pairs:
            def blk(k, _, eu=eu, ei=ei):
                off = (s + NSUB * k) * BLK
                de = pltpu.async_copy(eu.at[pl.ds(off, BLK)], eu_buf, seme)
                di = pltpu.async_copy(ei.at[pl.ds(off, BLK)], ei_buf, seme)
                de.wait()
                di.wait()
                for j in range(NJ):
                    for t in range(CH // 16):
                        sl = pl.ds(j * CH + t * 16, 16)
                        d16 = pl.ds(t * 16, 16)
                        u = eu_buf[sl]
                        i = ei_buf[sl]
                        gu2d[j, d16] = u + cN
                        su2d[j, d16] = u
                        gi2d[j, d16] = i + (NU + cN)
                        si2d[j, d16] = i + NU
                gds = []
                for j in range(NJ):
                    r = pl.ds(j * CH, CH)
                    gds.append(pltpu.async_copy(
                        table.at[gu2d.at[j]], rows_u.at[r], semg))
                    gds.append(pltpu.async_copy(
                        table.at[gi2d.at[j]], rows_i.at[r], semg))
                sds = []
                for j in range(NJ):
                    r = pl.ds(j * CH, CH)
                    gds[2 * j].wait()
                    sds.append(pltpu.async_copy(
                        rows_u.at[r], acc.at[si2d.at[j]], sems, add=True))
                    gds[2 * j + 1].wait()
                    sds.append(pltpu.async_copy(
                        rows_i.at[r], acc.at[su2d.at[j]], sems, add=True))
                for d in sds:
                    d.wait()
                return 0

            nk = (NBLK - s + NSUB - 1) // NSUB
            lax.fori_loop(0, nk, blk, 0)

        plsc.subcore_barrier()

        def ochunk(k, _):
            pltpu.sync_copy(acc.at[pl.ds(lo + k * ZR, ZR)],
                            rows_u.at[pl.ds(0, ZR)])
            pltpu.sync_copy(rows_u.at[pl.ds(0, ZR)],
                            out.at[pl.ds(c * NPAD + lo + k * ZR, ZR)])
            return 0

        lax.fori_loop(0, TROWS // ZR, ochunk, 0)

    return body


@functools.lru_cache(maxsize=None)
def _make_prop_kernel(n_edges):
    body = _make_prop_body(n_edges)
    return functools.partial(
        pl.kernel,
        out_type=jax.ShapeDtypeStruct((NCORE * NPAD, HD), jnp.float32),
        mesh=_mesh(),
        compiler_params=_SC_PARAMS,
        scratch_types=[
            pltpu.VMEM((BLK,), jnp.int32),
            pltpu.VMEM((BLK,), jnp.int32),
            pltpu.VMEM((NJ, CH), jnp.int32),
            pltpu.VMEM((NJ, CH), jnp.int32),
            pltpu.VMEM((NJ, CH), jnp.int32),
            pltpu.VMEM((NJ, CH), jnp.int32),
            pltpu.VMEM((BLK, HD), jnp.float32),
            pltpu.VMEM((BLK, HD), jnp.float32),
            pltpu.VMEM_SHARED((NPAD, HD), jnp.float32),
            pltpu.SemaphoreType.DMA,
            pltpu.SemaphoreType.DMA,
            pltpu.SemaphoreType.DMA,
        ],
    )(body)


def _gath_body(table, pairT, out, jbuf, gbuf, rows, sem):
    c = lax.axis_index("c")
    s = lax.axis_index("s")
    cN = c * N
    per_tile = B // NSUB
    for k in range(3):
        for q in range(per_tile // CH):
            off = s * per_tile + q * CH
            pltpu.sync_copy(pairT.at[k, pl.ds(off, CH)], jbuf)
            _build_idx(gbuf, jbuf, cN if k == 0 else cN + NU)
            pltpu.async_copy(table.at[gbuf], rows, sem).wait()
            base = c * (3 * B) + k * B + off
            pltpu.sync_copy(rows, out.at[pl.ds(base, CH)])


@functools.lru_cache(maxsize=None)
def _gath_kernel():
    return functools.partial(
        pl.kernel,
        out_type=jax.ShapeDtypeStruct((NCORE * 3 * B, HD), jnp.float32),
        mesh=_mesh(),
        compiler_params=_SC_PARAMS,
        scratch_types=[
            pltpu.VMEM((CH,), jnp.int32),
            pltpu.VMEM((CH,), jnp.int32),
            pltpu.VMEM((CH, HD), jnp.float32),
            pltpu.SemaphoreType.DMA,
        ],
    )(_gath_body)


NB = 2048
GN = -(-N // NB)


def _dinv_of(d):
    return jnp.where(d > 0.0, lax.rsqrt(jnp.maximum(d, 1.0)), 0.0)


def _prep_body(x0_ref, degp_ref, dinv_ref, z0_ref, y_ref):
    deg = degp_ref[...]
    degs = deg[0] + deg[1]
    degh = degs[0] + degs[1] + degs[2]
    dh = _dinv_of(degh)
    dinv_ref[0] = dh
    for b in range(3):
        dinv_ref[1 + b] = _dinv_of(degs[b])
    x = x0_ref[...]
    z0_ref[0] = x[:, :HD]
    z0_ref[1] = x[:, HD:]
    y_ref[0] = x[:, :HD] * dh
    y_ref[1] = x[:, HD:] * dh


def _tc_prep(x0, degp):
    return pl.pallas_call(
        _prep_body,
        grid=(GN,),
        in_specs=[
            pl.BlockSpec((NB, D), lambda j: (j, 0)),
            pl.BlockSpec((2, 3, NB, 1), lambda j: (0, 0, j, 0)),
        ],
        out_specs=[
            pl.BlockSpec((4, NB, 1), lambda j: (0, j, 0)),
            pl.BlockSpec((2, NB, HD), lambda j: (0, j, 0)),
            pl.BlockSpec((2, NB, HD), lambda j: (0, j, 0)),
        ],
        out_shape=[
            jax.ShapeDtypeStruct((4, N, 1), jnp.float32),
            jax.ShapeDtypeStruct((2, N, HD), jnp.float32),
            jax.ShapeDtypeStruct((2, N, HD), jnp.float32),
        ],
    )(x0, degp)


def _hdgfin_body(z0_ref, s_ref, dinv_ref, total_ref, y0_ref):
    dh = dinv_ref[0][None]
    d0 = dinv_ref[1][None]
    tot = 0.5 * (z0_ref[...] + s_ref[...] * dh)
    total_ref[...] = tot
    y0_ref[...] = tot * d0


def _tc_hdgfin(z0, s, dinv):
    return pl.pallas_call(
        _hdgfin_body,
        grid=(GN,),
        in_specs=[
            pl.BlockSpec((2, NB, HD), lambda j: (0, j, 0)),
            pl.BlockSpec((2, NB, HD), lambda j: (0, j, 0)),
            pl.BlockSpec((4, NB, 1), lambda j: (0, j, 0)),
        ],
        out_specs=[
            pl.BlockSpec((2, NB, HD), lambda j: (0, j, 0)),
            pl.BlockSpec((2, NB, HD), lambda j: (0, j, 0)),
        ],
        out_shape=[
            jax.ShapeDtypeStruct((2, N, HD), jnp.float32),
            jax.ShapeDtypeStruct((2, N, HD), jnp.float32),
        ],
    )(z0, s, dinv)


def _make_midscale_body(b):
    def body(s1_ref, dinv_ref, y1_ref):
        db = dinv_ref[1 + b][None]
        y1_ref[...] = s1_ref[...] * (db * db)
    return body


def _tc_midscale(s1, dinv, b):
    return pl.pallas_call(
        _make_midscale_body(b),
        grid=(GN,),
        in_specs=[
            pl.BlockSpec((2, NB, HD), lambda j: (0, j, 0)),
            pl.BlockSpec((4, NB, 1), lambda j: (0, j, 0)),
        ],
        out_specs=pl.BlockSpec((2, NB, HD), lambda j: (0, j, 0)),
        out_shape=jax.ShapeDtypeStruct((2, N, HD), jnp.float32),
    )(s1, dinv)


def _make_combine_body(b, last):
    def body(*refs):
        if b == 0:
            total_ref, s1_ref, s2_ref, dinv_ref, bw_ref = refs[:5]
            outs = refs[5:]
            acc_prev = None
        else:
            total_ref, s1_ref, s2_ref, dinv_ref, bw_ref, acc_ref = refs[:6]
            outs = refs[6:]
            acc_prev = acc_ref[...]
        if last:
            acc_out_ref, = outs
        else:
            total_out_ref, acc_out_ref, ynext_ref = outs

        db = dinv_ref[1 + b][None]
        total = total_ref[...]
        h1 = s1_ref[...] * db
        h2 = s2_ref[...] * db
        layer = (total + h1 + h2) * (1.0 / 3.0)
        ss = jnp.sum(layer * layer, axis=(0, 2))
        scale = (1.0 / jnp.maximum(jnp.sqrt(ss), 1e-12))[None, :, None]
        tot2 = total + layer * scale
        sw = 1.0 / (1.0 + jnp.exp(-bw_ref[b]))
        acc2 = sw * tot2 if acc_prev is None else acc_prev + sw * tot2
        acc_out_ref[...] = acc2
        if not last:
            total_out_ref[...] = tot2
            ynext_ref[...] = tot2 * dinv_ref[2 + b][None]
    return body


def _tc_combine(total, s1, s2, dinv, bw, acc_prev, b):
    last = b == 2
    blk = pl.BlockSpec((2, NB, HD), lambda j: (0, j, 0))
    in_specs = [
        blk, blk, blk,
        pl.BlockSpec((4, NB, 1), lambda j: (0, j, 0)),
        pl.BlockSpec(memory_space=pltpu.SMEM),
    ]
    args = [total, s1, s2, dinv, bw]
    if b > 0:
        in_specs.append(blk)
        args.append(acc_prev)
    if last:
        out_specs = [blk]
        out_shape = [jax.ShapeDtypeStruct((2, N, HD), jnp.float32)]
    else:
        out_specs = [blk, blk, blk]
        out_shape = [jax.ShapeDtypeStruct((2, N, HD), jnp.float32)] * 3
    res = pl.pallas_call(
        _make_combine_body(b, last),
        grid=(GN,),
        in_specs=in_specs,
        out_specs=out_specs,
        out_shape=out_shape,
    )(*args)
    if last:
        return None, res[0], None
    return res[0], res[1], res[2]


BJ = 1024
GJ = B // BJ


def _loss_body(g_ref, p_ref, s1_ref, s2_ref):
    j = pl.program_id(0)
    g = g_ref[...]
    u = g[:, 0]
    i1 = g[:, 1]
    i2 = g[:, 2]
    sp = jnp.sum(u * i1, axis=(0, 2))
    sn = jnp.sum(u * i2, axis=(0, 2))
    z = sp - sn
    vals = jnp.where(z > 0.0, -jnp.log1p(jnp.exp(-z)), z - jnp.log1p(jnp.exp(z)))
    m = jnp.any(p_ref[...] != 0, axis=0).astype(jnp.float32)

    @pl.when(j == 0)
    def _():
        s1_ref[...] = jnp.zeros((1, 1), jnp.float32)
        s2_ref[...] = jnp.zeros((1, 1), jnp.float32)

    s1_ref[...] += jnp.sum(vals * m).reshape(1, 1)
    s2_ref[...] += jnp.sum(m).reshape(1, 1)


def _tc_loss(gath, pairT):
    return pl.pallas_call(
        _loss_body,
        grid=(GJ,),
        in_specs=[
            pl.BlockSpec((2, 3, BJ, HD), lambda j: (0, 0, j, 0)),
            pl.BlockSpec((3, BJ), lambda j: (0, j)),
        ],
        out_specs=[
            pl.BlockSpec((1, 1), lambda j: (0, 0)),
            pl.BlockSpec((1, 1), lambda j: (0, 0)),
        ],
        out_shape=[
            jax.ShapeDtypeStruct((1, 1), jnp.float32),
            jax.ShapeDtypeStruct((1, 1), jnp.float32),
        ],
    )(gath, pairT)


BU = 2048
GU = -(-NU // BU)


def _frob_body(u_ref, i_ref, su_ref, si_ref):
    j = pl.program_id(0)
    rows = lax.broadcasted_iota(jnp.int32, (BU, D), 0) + j * BU
    msk = (rows < NU).astype(jnp.float32)
    u = u_ref[...] * msk
    v = i_ref[...] * msk

    @pl.when(j == 0)
    def _():
        su_ref[...] = jnp.zeros((1, 1), jnp.float32)
        si_ref[...] = jnp.zeros((1, 1), jnp.float32)

    su_ref[...] += jnp.sum(u * u).reshape(1, 1)
    si_ref[...] += jnp.sum(v * v).reshape(1, 1)


def _tc_frob(user_emb, item_emb):
    return pl.pallas_call(
        _frob_body,
        grid=(GU,),
        in_specs=[
            pl.BlockSpec((BU, D), lambda j: (j, 0)),
            pl.BlockSpec((BU, D), lambda j: (j, 0)),
        ],
        out_specs=[
            pl.BlockSpec((1, 1), lambda j: (0, 0)),
            pl.BlockSpec((1, 1), lambda j: (0, 0)),
        ],
        out_shape=[
            jax.ShapeDtypeStruct((1, 1), jnp.float32),
            jax.ShapeDtypeStruct((1, 1), jnp.float32),
        ],
    )(user_emb, item_emb)


def kernel(user_emb, item_emb, behavior_weights, batch_data,
           edges_view, edges_cart, edges_buy):
    f32 = jnp.float32
    _prop1 = _make_prop_kernel(1)
    _prop3 = _make_prop_kernel(3)
    eu = [e[0] for e in (edges_view, edges_cart, edges_buy)]
    ei = [e[1] for e in (edges_view, edges_cart, edges_buy)]
    zeros_deg = jnp.zeros((DROWS,), f32)
    zeros_prop = jnp.zeros((ZR, HD), f32)
    x0 = jnp.concatenate([user_emb, item_emb], axis=0)

    degp = _deg_kernel()(eu[0], ei[0], eu[1], ei[1], eu[2], ei[2], zeros_deg)
    degp = degp.reshape(2, DPAD)[:, :3 * N].reshape(2, 3, N, 1)
    dinv, z0, y = _tc_prep(x0, degp)

    s = _prop3(y.reshape(2 * N, HD), eu[0], ei[0], eu[1], ei[1],
               eu[2], ei[2], zeros_prop)
    total, ynext = _tc_hdgfin(z0, s.reshape(2, NPAD, HD), dinv)

    acc = None
    for b in range(3):
        s1 = _prop1(ynext.reshape(2 * N, HD), eu[b], ei[b], zeros_prop)
        s1 = s1.reshape(2, NPAD, HD)
        y1 = _tc_midscale(s1, dinv, b)
        s2 = _prop1(y1.reshape(2 * N, HD), eu[b], ei[b], zeros_prop)
        s2 = s2.reshape(2, NPAD, HD)
        total, acc, ynext = _tc_combine(total, s1, s2, dinv,
                                        behavior_weights, acc, b)

    pairT = batch_data[:, -1, :3].T
    gath = _gath_kernel()(acc.reshape(2 * N, HD), pairT)
    s1_, s2_ = _tc_loss(gath.reshape(2, 3, B, HD), pairT)
    su, si = _tc_frob(user_emb, item_emb)

    bpr = -s1_[0, 0] / s2_[0, 0]
    emb = (jnp.sqrt(su[0, 0]) + jnp.sqrt(si[0, 0])) / (N_ITEMS + 1)
    return bpr + REG_WEIGHT * emb


# speedup vs baseline: 38.7248x; 1.1678x over previous
import functools

import jax
import jax.numpy as jnp
from jax import lax
from jax.experimental import pallas as pl
from jax.experimental.pallas import tpu as pltpu
from jax.experimental.pallas import tpu_sc as plsc

N_USERS = 25000
N_ITEMS = 25000
D = 64
E = 800000
N_BEH = 3
B = 4096
REG_WEIGHT = 1e-3
NU = N_USERS + 1
N = NU + (N_ITEMS + 1)
HD = D // 2

CH = 128
NCHUNK = E // CH
NSUB = 16
NCORE = 2

NPAD = 51200
TROWS = NPAD // NSUB
ZR = CH
BLK = 256
NJ = BLK // CH
NBLK = E // BLK
DPAD = 150016
DROWS = DPAD // NSUB

@functools.lru_cache(maxsize=None)
def _mesh():
    return plsc.VectorSubcoreMesh(core_axis_name="c", subcore_axis_name="s",
                                  num_cores=NCORE, num_subcores=NSUB)


_SC_PARAMS = pltpu.CompilerParams(use_tc_tiling_on_sc=False)


def _build_idx(dst_ref, src_ref, offset):
    for t in range(CH // 16):
        sl = pl.ds(t * 16, 16)
        dst_ref[sl] = src_ref[sl] + offset


def _deg_body(eu0, ei0, eu1, ei1, eu2, ei2, zeros, out,
              u_buf, i_buf, gu_buf, gi_buf, ones, vbuf, acc, sem):
    c = lax.axis_index("c")
    s = lax.axis_index("s")
    wid = s * NCORE + c

    for t in range(CH // 16):
        ones[pl.ds(t * 16, 16)] = jnp.full((16,), 1.0, jnp.float32)

    lo = s * DROWS
    pltpu.sync_copy(zeros, vbuf)
    pltpu.sync_copy(vbuf, acc.at[pl.ds(lo, DROWS)])

    plsc.subcore_barrier()

    for b, (eu, ei) in enumerate(((eu0, ei0), (eu1, ei1), (eu2, ei2))):
        bN = b * N

        def chunk(k, _, eu=eu, ei=ei, bN=bN):
            off = (wid + 32 * k) * CH
            pltpu.sync_copy(eu.at[pl.ds(off, CH)], u_buf)
            pltpu.sync_copy(ei.at[pl.ds(off, CH)], i_buf)
            _build_idx(gu_buf, u_buf, bN)
            _build_idx(gi_buf, i_buf, bN + NU)
            pltpu.sync_copy(ones, acc.at[gu_buf], add=True)
            pltpu.sync_copy(ones, acc.at[gi_buf], add=True)
            return 0

        nk = (NCHUNK - wid + 31) // 32
        lax.fori_loop(0, nk, chunk, 0)

    plsc.subcore_barrier()

    pltpu.sync_copy(acc.at[pl.ds(lo, DROWS)], vbuf)
    pltpu.sync_copy(vbuf, out.at[pl.ds(c * DPAD + lo, DROWS)])


@functools.lru_cache(maxsize=None)
def _deg_kernel():
    return functools.partial(
        pl.kernel,
        out_type=jax.ShapeDtypeStruct((NCORE * DPAD,), jnp.float32),
        mesh=_mesh(),
        compiler_params=_SC_PARAMS,
        scratch_types=[
            pltpu.VMEM((CH,), jnp.int32),
            pltpu.VMEM((CH,), jnp.int32),
            pltpu.VMEM((CH,), jnp.int32),
            pltpu.VMEM((CH,), jnp.int32),
            pltpu.VMEM((CH,), jnp.float32),
            pltpu.VMEM((DROWS,), jnp.float32),
            pltpu.VMEM_SHARED((DPAD,), jnp.float32),
            pltpu.SemaphoreType.DMA,
        ],
    )(_deg_body)


def _make_prop_body(n_edges):
    def body(*refs):
        table = refs[0]
        pairs = [(refs[1 + 2 * j], refs[2 + 2 * j]) for j in range(n_edges)]
        zeros = refs[1 + 2 * n_edges]
        out = refs[2 + 2 * n_edges]
        (eu_buf, ei_buf, gu2d, su2d, gi2d, si2d, rows_u, rows_i,
         acc, semg, sems, seme) = refs[3 + 2 * n_edges:]

        c = lax.axis_index("c")
        s = lax.axis_index("s")
        cN = c * N

        lo = s * TROWS
        pltpu.sync_copy(zeros, rows_u.at[pl.ds(0, ZR)])

        def zchunk(k, _):
            pltpu.sync_copy(rows_u.at[pl.ds(0, ZR)],
                            acc.at[pl.ds(lo + k * ZR, ZR)])
            return 0

        lax.fori_loop(0, TROWS // ZR, zchunk, 0)

        plsc.subcore_barrier()

        def drain_scatters():
            for j in range(NJ):
                r = pl.ds(j * CH, CH)
                pltpu.make_async_copy(
                    rows_u.at[r], acc.at[si2d.at[j]], sems).wait()
                pltpu.make_async_copy(
                    rows_i.at[r], acc.at[su2d.at[j]], sems).wait()

        for eu, ei in pairs:
            nk = (NBLK - s + NSUB - 1) // NSUB

            def fire_edges(k, eu=eu, ei=ei):
                off = (s + NSUB * k) * BLK
                pltpu.async_copy(eu.at[pl.ds(off, BLK)], eu_buf, seme)
                pltpu.async_copy(ei.at[pl.ds(off, BLK)], ei_buf, seme)

            @pl.when(nk > 0)
            def _():
                fire_edges(0)

            def blk(k, _, eu=eu, ei=ei):
                @pl.when(k > 0)
                def _():
                    drain_scatters()

                off = (s + NSUB * k) * BLK
                pltpu.make_async_copy(
                    eu.at[pl.ds(off, BLK)], eu_buf, seme).wait()
                pltpu.make_async_copy(
                    ei.at[pl.ds(off, BLK)], ei_buf, seme).wait()
                for j in range(NJ):
                    for t in range(CH // 16):
                        sl = pl.ds(j * CH + t * 16, 16)
                        d16 = pl.ds(t * 16, 16)
                        u = eu_buf[sl]
                        i = ei_buf[sl]
                        gu2d[j, d16] = u + cN
                        su2d[j, d16] = u
                        gi2d[j, d16] = i + (NU + cN)
                        si2d[j, d16] = i + NU

                @pl.when(k + 1 < nk)
                def _():
                    fire_edges_dyn(k + 1)

                gds = []
                for j in range(NJ):
                    r = pl.ds(j * CH, CH)
                    gds.append(pltpu.async_copy(
                        table.at[gu2d.at[j]], rows_u.at[r], semg))
                    gds.append(pltpu.async_copy(
                        table.at[gi2d.at[j]], rows_i.at[r], semg))
                for j in range(NJ):
                    r = pl.ds(j * CH, CH)
                    gds[2 * j].wait()
                    pltpu.async_copy(
                        rows_u.at[r], acc.at[si2d.at[j]], sems, add=True)
                    gds[2 * j + 1].wait()
                    pltpu.async_copy(
                        rows_i.at[r], acc.at[su2d.at[j]], sems, add=True)
                return 0

            def fire_edges_dyn(k1, eu=eu, ei=ei):
                off = (s + NSUB * k1) * BLK
                pltpu.async_copy(eu.at[pl.ds(off, BLK)], eu_buf, seme)
                pltpu.async_copy(ei.at[pl.ds(off, BLK)], ei_buf, seme)

            lax.fori_loop(0, nk, blk, 0)

            @pl.when(nk > 0)
            def _():
                drain_scatters()

        plsc.subcore_barrier()

        def ochunk(k, _):
            pltpu.sync_copy(acc.at[pl.ds(lo + k * ZR, ZR)],
                            rows_u.at[pl.ds(0, ZR)])
            pltpu.sync_copy(rows_u.at[pl.ds(0, ZR)],
                            out.at[pl.ds(c * NPAD + lo + k * ZR, ZR)])
            return 0

        lax.fori_loop(0, TROWS // ZR, ochunk, 0)

    return body


@functools.lru_cache(maxsize=None)
def _make_prop_kernel(n_edges):
    body = _make_prop_body(n_edges)
    return functools.partial(
        pl.kernel,
        out_type=jax.ShapeDtypeStruct((NCORE * NPAD, HD), jnp.float32),
        mesh=_mesh(),
        compiler_params=_SC_PARAMS,
        scratch_types=[
            pltpu.VMEM((BLK,), jnp.int32),
            pltpu.VMEM((BLK,), jnp.int32),
            pltpu.VMEM((NJ, CH), jnp.int32),
            pltpu.VMEM((NJ, CH), jnp.int32),
            pltpu.VMEM((NJ, CH), jnp.int32),
            pltpu.VMEM((NJ, CH), jnp.int32),
            pltpu.VMEM((BLK, HD), jnp.float32),
            pltpu.VMEM((BLK, HD), jnp.float32),
            pltpu.VMEM_SHARED((NPAD, HD), jnp.float32),
            pltpu.SemaphoreType.DMA,
            pltpu.SemaphoreType.DMA,
            pltpu.SemaphoreType.DMA,
        ],
    )(body)


def _gath_body(table, pairT, out, jbuf, gbuf, rows, sem):
    c = lax.axis_index("c")
    s = lax.axis_index("s")
    cN = c * N
    per_tile = B // NSUB
    for k in range(3):
        for q in range(per_tile // CH):
            off = s * per_tile + q * CH
            pltpu.sync_copy(pairT.at[k, pl.ds(off, CH)], jbuf)
            _build_idx(gbuf, jbuf, cN if k == 0 else cN + NU)
            pltpu.async_copy(table.at[gbuf], rows, sem).wait()
            base = c * (3 * B) + k * B + off
            pltpu.sync_copy(rows, out.at[pl.ds(base, CH)])


@functools.lru_cache(maxsize=None)
def _gath_kernel():
    return functools.partial(
        pl.kernel,
        out_type=jax.ShapeDtypeStruct((NCORE * 3 * B, HD), jnp.float32),
        mesh=_mesh(),
        compiler_params=_SC_PARAMS,
        scratch_types=[
            pltpu.VMEM((CH,), jnp.int32),
            pltpu.VMEM((CH,), jnp.int32),
            pltpu.VMEM((CH, HD), jnp.float32),
            pltpu.SemaphoreType.DMA,
        ],
    )(_gath_body)


NB = 2048
GN = -(-N // NB)


def _dinv_of(d):
    return jnp.where(d > 0.0, lax.rsqrt(jnp.maximum(d, 1.0)), 0.0)


def _prep_body(x0_ref, degp_ref, dinv_ref, z0_ref, y_ref):
    deg = degp_ref[...]
    degs = deg[0] + deg[1]
    degh = degs[0] + degs[1] + degs[2]
    dh = _dinv_of(degh)
    dinv_ref[0] = dh
    for b in range(3):
        dinv_ref[1 + b] = _dinv_of(degs[b])
    x = x0_ref[...]
    z0_ref[0] = x[:, :HD]
    z0_ref[1] = x[:, HD:]
    y_ref[0] = x[:, :HD] * dh
    y_ref[1] = x[:, HD:] * dh


def _tc_prep(x0, degp):
    return pl.pallas_call(
        _prep_body,
        grid=(GN,),
        in_specs=[
            pl.BlockSpec((NB, D), lambda j: (j, 0)),
            pl.BlockSpec((2, 3, NB, 1), lambda j: (0, 0, j, 0)),
        ],
        out_specs=[
            pl.BlockSpec((4, NB, 1), lambda j: (0, j, 0)),
            pl.BlockSpec((2, NB, HD), lambda j: (0, j, 0)),
            pl.BlockSpec((2, NB, HD), lambda j: (0, j, 0)),
        ],
        out_shape=[
            jax.ShapeDtypeStruct((4, N, 1), jnp.float32),
            jax.ShapeDtypeStruct((2, N, HD), jnp.float32),
            jax.ShapeDtypeStruct((2, N, HD), jnp.float32),
        ],
    )(x0, degp)


def _hdgfin_body(z0_ref, s_ref, dinv_ref, total_ref, y0_ref):
    dh = dinv_ref[0][None]
    d0 = dinv_ref[1][None]
    tot = 0.5 * (z0_ref[...] + s_ref[...] * dh)
    total_ref[...] = tot
    y0_ref[...] = tot * d0


def _tc_hdgfin(z0, s, dinv):
    return pl.pallas_call(
        _hdgfin_body,
        grid=(GN,),
        in_specs=[
            pl.BlockSpec((2, NB, HD), lambda j: (0, j, 0)),
            pl.BlockSpec((2, NB, HD), lambda j: (0, j, 0)),
            pl.BlockSpec((4, NB, 1), lambda j: (0, j, 0)),
        ],
        out_specs=[
            pl.BlockSpec((2, NB, HD), lambda j: (0, j, 0)),
            pl.BlockSpec((2, NB, HD), lambda j: (0, j, 0)),
        ],
        out_shape=[
            jax.ShapeDtypeStruct((2, N, HD), jnp.float32),
            jax.ShapeDtypeStruct((2, N, HD), jnp.float32),
        ],
    )(z0, s, dinv)


def _make_midscale_body(b):
    def body(s1_ref, dinv_ref, y1_ref):
        db = dinv_ref[1 + b][None]
        y1_ref[...] = s1_ref[...] * (db * db)
    return body


def _tc_midscale(s1, dinv, b):
    return pl.pallas_call(
        _make_midscale_body(b),
        grid=(GN,),
        in_specs=[
            pl.BlockSpec((2, NB, HD), lambda j: (0, j, 0)),
            pl.BlockSpec((4, NB, 1), lambda j: (0, j, 0)),
        ],
        out_specs=pl.BlockSpec((2, NB, HD), lambda j: (0, j, 0)),
        out_shape=jax.ShapeDtypeStruct((2, N, HD), jnp.float32),
    )(s1, dinv)


def _make_combine_body(b, last):
    def body(*refs):
        if b == 0:
            total_ref, s1_ref, s2_ref, dinv_ref, bw_ref = refs[:5]
            outs = refs[5:]
            acc_prev = None
        else:
            total_ref, s1_ref, s2_ref, dinv_ref, bw_ref, acc_ref = refs[:6]
            outs = refs[6:]
            acc_prev = acc_ref[...]
        if last:
            acc_out_ref, = outs
        else:
            total_out_ref, acc_out_ref, ynext_ref = outs

        db = dinv_ref[1 + b][None]
        total = total_ref[...]
        h1 = s1_ref[...] * db
        h2 = s2_ref[...] * db
        layer = (total + h1 + h2) * (1.0 / 3.0)
        ss = jnp.sum(layer * layer, axis=(0, 2))
        scale = (1.0 / jnp.maximum(jnp.sqrt(ss), 1e-12))[None, :, None]
        tot2 = total + layer * scale
        sw = 1.0 / (1.0 + jnp.exp(-bw_ref[b]))
        acc2 = sw * tot2 if acc_prev is None else acc_prev + sw * tot2
        acc_out_ref[...] = acc2
        if not last:
            total_out_ref[...] = tot2
            ynext_ref[...] = tot2 * dinv_ref[2 + b][None]
    return body


def _tc_combine(total, s1, s2, dinv, bw, acc_prev, b):
    last = b == 2
    blk = pl.BlockSpec((2, NB, HD), lambda j: (0, j, 0))
    in_specs = [
        blk, blk, blk,
        pl.BlockSpec((4, NB, 1), lambda j: (0, j, 0)),
        pl.BlockSpec(memory_space=pltpu.SMEM),
    ]
    args = [total, s1, s2, dinv, bw]
    if b > 0:
        in_specs.append(blk)
        args.append(acc_prev)
    if last:
        out_specs = [blk]
        out_shape = [jax.ShapeDtypeStruct((2, N, HD), jnp.float32)]
    else:
        out_specs = [blk, blk, blk]
        out_shape = [jax.ShapeDtypeStruct((2, N, HD), jnp.float32)] * 3
    res = pl.pallas_call(
        _make_combine_body(b, last),
        grid=(GN,),
        in_specs=in_specs,
        out_specs=out_specs,
        out_shape=out_shape,
    )(*args)
    if last:
        return None, res[0], None
    return res[0], res[1], res[2]


BJ = 1024
GJ = B // BJ


def _loss_body(g_ref, p_ref, s1_ref, s2_ref):
    j = pl.program_id(0)
    g = g_ref[...]
    u = g[:, 0]
    i1 = g[:, 1]
    i2 = g[:, 2]
    sp = jnp.sum(u * i1, axis=(0, 2))
    sn = jnp.sum(u * i2, axis=(0, 2))
    z = sp - sn
    vals = jnp.where(z > 0.0, -jnp.log1p(jnp.exp(-z)), z - jnp.log1p(jnp.exp(z)))
    m = jnp.any(p_ref[...] != 0, axis=0).astype(jnp.float32)

    @pl.when(j == 0)
    def _():
        s1_ref[...] = jnp.zeros((1, 1), jnp.float32)
        s2_ref[...] = jnp.zeros((1, 1), jnp.float32)

    s1_ref[...] += jnp.sum(vals * m).reshape(1, 1)
    s2_ref[...] += jnp.sum(m).reshape(1, 1)


def _tc_loss(gath, pairT):
    return pl.pallas_call(
        _loss_body,
        grid=(GJ,),
        in_specs=[
            pl.BlockSpec((2, 3, BJ, HD), lambda j: (0, 0, j, 0)),
            pl.BlockSpec((3, BJ), lambda j: (0, j)),
        ],
        out_specs=[
            pl.BlockSpec((1, 1), lambda j: (0, 0)),
            pl.BlockSpec((1, 1), lambda j: (0, 0)),
        ],
        out_shape=[
            jax.ShapeDtypeStruct((1, 1), jnp.float32),
            jax.ShapeDtypeStruct((1, 1), jnp.float32),
        ],
    )(gath, pairT)


BU = 2048
GU = -(-NU // BU)


def _frob_body(u_ref, i_ref, su_ref, si_ref):
    j = pl.program_id(0)
    rows = lax.broadcasted_iota(jnp.int32, (BU, D), 0) + j * BU
    msk = (rows < NU).astype(jnp.float32)
    u = u_ref[...] * msk
    v = i_ref[...] * msk

    @pl.when(j == 0)
    def _():
        su_ref[...] = jnp.zeros((1, 1), jnp.float32)
        si_ref[...] = jnp.zeros((1, 1), jnp.float32)

    su_ref[...] += jnp.sum(u * u).reshape(1, 1)
    si_ref[...] += jnp.sum(v * v).reshape(1, 1)


def _tc_frob(user_emb, item_emb):
    return pl.pallas_call(
        _frob_body,
        grid=(GU,),
        in_specs=[
            pl.BlockSpec((BU, D), lambda j: (j, 0)),
            pl.BlockSpec((BU, D), lambda j: (j, 0)),
        ],
        out_specs=[
            pl.BlockSpec((1, 1), lambda j: (0, 0)),
            pl.BlockSpec((1, 1), lambda j: (0, 0)),
        ],
        out_shape=[
            jax.ShapeDtypeStruct((1, 1), jnp.float32),
            jax.ShapeDtypeStruct((1, 1), jnp.float32),
        ],
    )(user_emb, item_emb)


def kernel(user_emb, item_emb, behavior_weights, batch_data,
           edges_view, edges_cart, edges_buy):
    f32 = jnp.float32
    _prop1 = _make_prop_kernel(1)
    _prop3 = _make_prop_kernel(3)
    eu = [e[0] for e in (edges_view, edges_cart, edges_buy)]
    ei = [e[1] for e in (edges_view, edges_cart, edges_buy)]
    zeros_deg = jnp.zeros((DROWS,), f32)
    zeros_prop = jnp.zeros((ZR, HD), f32)
    x0 = jnp.concatenate([user_emb, item_emb], axis=0)

    degp = _deg_kernel()(eu[0], ei[0], eu[1], ei[1], eu[2], ei[2], zeros_deg)
    degp = degp.reshape(2, DPAD)[:, :3 * N].reshape(2, 3, N, 1)
    dinv, z0, y = _tc_prep(x0, degp)

    s = _prop3(y.reshape(2 * N, HD), eu[0], ei[0], eu[1], ei[1],
               eu[2], ei[2], zeros_prop)
    total, ynext = _tc_hdgfin(z0, s.reshape(2, NPAD, HD), dinv)

    acc = None
    for b in range(3):
        s1 = _prop1(ynext.reshape(2 * N, HD), eu[b], ei[b], zeros_prop)
        s1 = s1.reshape(2, NPAD, HD)
        y1 = _tc_midscale(s1, dinv, b)
        s2 = _prop1(y1.reshape(2 * N, HD), eu[b], ei[b], zeros_prop)
        s2 = s2.reshape(2, NPAD, HD)
        total, acc, ynext = _tc_combine(total, s1, s2, dinv,
                                        behavior_weights, acc, b)

    pairT = batch_data[:, -1, :3].T
    gath = _gath_kernel()(acc.reshape(2 * N, HD), pairT)
    s1_, s2_ = _tc_loss(gath.reshape(2, 3, B, HD), pairT)
    su, si = _tc_frob(user_emb, item_emb)

    bpr = -s1_[0, 0] / s2_[0, 0]
    emb = (jnp.sqrt(su[0, 0]) + jnp.sqrt(si[0, 0])) / (N_ITEMS + 1)
    return bpr + REG_WEIGHT * emb


# speedup vs baseline: 42.7670x; 1.1044x over previous
import functools

import jax
import jax.numpy as jnp
from jax import lax
from jax.experimental import pallas as pl
from jax.experimental.pallas import tpu as pltpu
from jax.experimental.pallas import tpu_sc as plsc

N_USERS = 25000
N_ITEMS = 25000
D = 64
E = 800000
N_BEH = 3
B = 4096
REG_WEIGHT = 1e-3
NU = N_USERS + 1
N = NU + (N_ITEMS + 1)
HD = D // 2

CH = 128
NCHUNK = E // CH
NSUB = 16
NCORE = 2

NPAD = 51200
TROWS = NPAD // NSUB
ZR = CH
BLK = 256
NJ = BLK // CH
NBLK = E // BLK
DPAD = 150016
DROWS = DPAD // NSUB

@functools.lru_cache(maxsize=None)
def _mesh():
    return plsc.VectorSubcoreMesh(core_axis_name="c", subcore_axis_name="s",
                                  num_cores=NCORE, num_subcores=NSUB)


_SC_PARAMS = pltpu.CompilerParams(use_tc_tiling_on_sc=False)


def _build_idx(dst_ref, src_ref, offset):
    for t in range(CH // 16):
        sl = pl.ds(t * 16, 16)
        dst_ref[sl] = src_ref[sl] + offset


def _deg_body(eu0, ei0, eu1, ei1, eu2, ei2, zeros, out,
              eu_buf, ei_buf, gu2d, gi2d, ones, vbuf, acc, seme, sems):
    c = lax.axis_index("c")
    s = lax.axis_index("s")
    wid = s * NCORE + c

    for t in range(CH // 16):
        ones[pl.ds(t * 16, 16)] = jnp.full((16,), 1.0, jnp.float32)

    lo = s * DROWS
    pltpu.sync_copy(zeros, vbuf)
    pltpu.sync_copy(vbuf, acc.at[pl.ds(lo, DROWS)])

    plsc.subcore_barrier()

    def drain_scatters():
        for j in range(NJ):
            pltpu.make_async_copy(ones, acc.at[gu2d.at[j]], sems).wait()
            pltpu.make_async_copy(ones, acc.at[gi2d.at[j]], sems).wait()

    for b, (eu, ei) in enumerate(((eu0, ei0), (eu1, ei1), (eu2, ei2))):
        bN = b * N
        nk = (NBLK - wid + 31) // 32

        def fire_edges(k1, eu=eu, ei=ei):
            off = (wid + 32 * k1) * BLK
            pltpu.async_copy(eu.at[pl.ds(off, BLK)], eu_buf, seme)
            pltpu.async_copy(ei.at[pl.ds(off, BLK)], ei_buf, seme)

        @pl.when(nk > 0)
        def _():
            fire_edges(0)

        def blk(k, _, eu=eu, ei=ei, bN=bN, fire_edges=fire_edges, nk=nk):
            @pl.when(k > 0)
            def _():
                drain_scatters()

            off = (wid + 32 * k) * BLK
            pltpu.make_async_copy(eu.at[pl.ds(off, BLK)], eu_buf, seme).wait()
            pltpu.make_async_copy(ei.at[pl.ds(off, BLK)], ei_buf, seme).wait()
            for j in range(NJ):
                for t in range(CH // 16):
                    sl = pl.ds(j * CH + t * 16, 16)
                    d16 = pl.ds(t * 16, 16)
                    gu2d[j, d16] = eu_buf[sl] + bN
                    gi2d[j, d16] = ei_buf[sl] + (bN + NU)

            @pl.when(k + 1 < nk)
            def _():
                fire_edges(k + 1)

            for j in range(NJ):
                pltpu.async_copy(ones, acc.at[gu2d.at[j]], sems, add=True)
                pltpu.async_copy(ones, acc.at[gi2d.at[j]], sems, add=True)
            return 0

        lax.fori_loop(0, nk, blk, 0)

        @pl.when(nk > 0)
        def _():
            drain_scatters()

    plsc.subcore_barrier()

    pltpu.sync_copy(acc.at[pl.ds(lo, DROWS)], vbuf)
    pltpu.sync_copy(vbuf, out.at[pl.ds(c * DPAD + lo, DROWS)])


@functools.lru_cache(maxsize=None)
def _deg_kernel():
    return functools.partial(
        pl.kernel,
        out_type=jax.ShapeDtypeStruct((NCORE * DPAD,), jnp.float32),
        mesh=_mesh(),
        compiler_params=_SC_PARAMS,
        scratch_types=[
            pltpu.VMEM((BLK,), jnp.int32),
            pltpu.VMEM((BLK,), jnp.int32),
            pltpu.VMEM((NJ, CH), jnp.int32),
            pltpu.VMEM((NJ, CH), jnp.int32),
            pltpu.VMEM((CH,), jnp.float32),
            pltpu.VMEM((DROWS,), jnp.float32),
            pltpu.VMEM_SHARED((DPAD,), jnp.float32),
            pltpu.SemaphoreType.DMA,
            pltpu.SemaphoreType.DMA,
        ],
    )(_deg_body)


def _make_prop_body(n_edges):
    def body(*refs):
        table = refs[0]
        pairs = [(refs[1 + 2 * j], refs[2 + 2 * j]) for j in range(n_edges)]
        zeros = refs[1 + 2 * n_edges]
        out = refs[2 + 2 * n_edges]
        (eu_buf, ei_buf, gu2d, su2d, gi2d, si2d, rows_u, rows_i,
         acc, semg, sems, seme) = refs[3 + 2 * n_edges:]

        c = lax.axis_index("c")
        s = lax.axis_index("s")
        cN = c * N

        lo = s * TROWS
        pltpu.sync_copy(zeros, rows_u.at[pl.ds(0, ZR)])

        def zchunk(k, _):
            pltpu.sync_copy(rows_u.at[pl.ds(0, ZR)],
                            acc.at[pl.ds(lo + k * ZR, ZR)])
            return 0

        lax.fori_loop(0, TROWS // ZR, zchunk, 0)

        plsc.subcore_barrier()

        def drain_scatters():
            for j in range(NJ):
                r = pl.ds(j * CH, CH)
                pltpu.make_async_copy(
                    rows_u.at[r], acc.at[si2d.at[j]], sems).wait()
                pltpu.make_async_copy(
                    rows_i.at[r], acc.at[su2d.at[j]], sems).wait()

        for eu, ei in pairs:
            nk = (NBLK - s + NSUB - 1) // NSUB

            def fire_edges(k, eu=eu, ei=ei):
                off = (s + NSUB * k) * BLK
                pltpu.async_copy(eu.at[pl.ds(off, BLK)], eu_buf, seme)
                pltpu.async_copy(ei.at[pl.ds(off, BLK)], ei_buf, seme)

            @pl.when(nk > 0)
            def _():
                fire_edges(0)

            def blk(k, _, eu=eu, ei=ei):
                @pl.when(k > 0)
                def _():
                    drain_scatters()

                off = (s + NSUB * k) * BLK
                pltpu.make_async_copy(
                    eu.at[pl.ds(off, BLK)], eu_buf, seme).wait()
                pltpu.make_async_copy(
                    ei.at[pl.ds(off, BLK)], ei_buf, seme).wait()
                for j in range(NJ):
                    for t in range(CH // 16):
                        sl = pl.ds(j * CH + t * 16, 16)
                        d16 = pl.ds(t * 16, 16)
                        u = eu_buf[sl]
                        i = ei_buf[sl]
                        gu2d[j, d16] = u + cN
                        su2d[j, d16] = u
                        gi2d[j, d16] = i + (NU + cN)
                        si2d[j, d16] = i + NU

                @pl.when(k + 1 < nk)
                def _():
                    fire_edges_dyn(k + 1)

                gds = []
                for j in range(NJ):
                    r = pl.ds(j * CH, CH)
                    gds.append(pltpu.async_copy(
                        table.at[gu2d.at[j]], rows_u.at[r], semg))
                    gds.append(pltpu.async_copy(
                        table.at[gi2d.at[j]], rows_i.at[r], semg))
                for j in range(NJ):
                    r = pl.ds(j * CH, CH)
                    gds[2 * j].wait()
                    pltpu.async_copy(
                        rows_u.at[r], acc.at[si2d.at[j]], sems, add=True)
                    gds[2 * j + 1].wait()
                    pltpu.async_copy(
                        rows_i.at[r], acc.at[su2d.at[j]], sems, add=True)
                return 0

            def fire_edges_dyn(k1, eu=eu, ei=ei):
                off = (s + NSUB * k1) * BLK
                pltpu.async_copy(eu.at[pl.ds(off, BLK)], eu_buf, seme)
                pltpu.async_copy(ei.at[pl.ds(off, BLK)], ei_buf, seme)

            lax.fori_loop(0, nk, blk, 0)

            @pl.when(nk > 0)
            def _():
                drain_scatters()

        plsc.subcore_barrier()

        def ochunk(k, _):
            pltpu.sync_copy(acc.at[pl.ds(lo + k * ZR, ZR)],
                            rows_u.at[pl.ds(0, ZR)])
            pltpu.sync_copy(rows_u.at[pl.ds(0, ZR)],
                            out.at[pl.ds(c * NPAD + lo + k * ZR, ZR)])
            return 0

        lax.fori_loop(0, TROWS // ZR, ochunk, 0)

    return body


@functools.lru_cache(maxsize=None)
def _make_prop_kernel(n_edges):
    body = _make_prop_body(n_edges)
    return functools.partial(
        pl.kernel,
        out_type=jax.ShapeDtypeStruct((NCORE * NPAD, HD), jnp.float32),
        mesh=_mesh(),
        compiler_params=_SC_PARAMS,
        scratch_types=[
            pltpu.VMEM((BLK,), jnp.int32),
            pltpu.VMEM((BLK,), jnp.int32),
            pltpu.VMEM((NJ, CH), jnp.int32),
            pltpu.VMEM((NJ, CH), jnp.int32),
            pltpu.VMEM((NJ, CH), jnp.int32),
            pltpu.VMEM((NJ, CH), jnp.int32),
            pltpu.VMEM((BLK, HD), jnp.float32),
            pltpu.VMEM((BLK, HD), jnp.float32),
            pltpu.VMEM_SHARED((NPAD, HD), jnp.float32),
            pltpu.SemaphoreType.DMA,
            pltpu.SemaphoreType.DMA,
            pltpu.SemaphoreType.DMA,
        ],
    )(body)


def _gath_body(table, pairT, out, jbuf, gbuf, rows, sem):
    c = lax.axis_index("c")
    s = lax.axis_index("s")
    cN = c * N
    per_tile = B // NSUB
    for k in range(3):
        for q in range(per_tile // CH):
            off = s * per_tile + q * CH
            pltpu.sync_copy(pairT.at[k, pl.ds(off, CH)], jbuf)
            _build_idx(gbuf, jbuf, cN if k == 0 else cN + NU)
            pltpu.async_copy(table.at[gbuf], rows, sem).wait()
            base = c * (3 * B) + k * B + off
            pltpu.sync_copy(rows, out.at[pl.ds(base, CH)])


@functools.lru_cache(maxsize=None)
def _gath_kernel():
    return functools.partial(
        pl.kernel,
        out_type=jax.ShapeDtypeStruct((NCORE * 3 * B, HD), jnp.float32),
        mesh=_mesh(),
        compiler_params=_SC_PARAMS,
        scratch_types=[
            pltpu.VMEM((CH,), jnp.int32),
            pltpu.VMEM((CH,), jnp.int32),
            pltpu.VMEM((CH, HD), jnp.float32),
            pltpu.SemaphoreType.DMA,
        ],
    )(_gath_body)


NB = 2048
GN = -(-N // NB)


def _dinv_of(d):
    return jnp.where(d > 0.0, lax.rsqrt(jnp.maximum(d, 1.0)), 0.0)


def _prep_body(x0_ref, degp_ref, dinv_ref, z0_ref, y_ref):
    deg = degp_ref[...]
    degs = deg[0] + deg[1]
    degh = degs[0] + degs[1] + degs[2]
    dh = _dinv_of(degh)
    dinv_ref[0] = dh
    for b in range(3):
        dinv_ref[1 + b] = _dinv_of(degs[b])
    x = x0_ref[...]
    z0_ref[0] = x[:, :HD]
    z0_ref[1] = x[:, HD:]
    y_ref[0] = x[:, :HD] * dh
    y_ref[1] = x[:, HD:] * dh


def _tc_prep(x0, degp):
    return pl.pallas_call(
        _prep_body,
        grid=(GN,),
        in_specs=[
            pl.BlockSpec((NB, D), lambda j: (j, 0)),
            pl.BlockSpec((2, 3, NB, 1), lambda j: (0, 0, j, 0)),
        ],
        out_specs=[
            pl.BlockSpec((4, NB, 1), lambda j: (0, j, 0)),
            pl.BlockSpec((2, NB, HD), lambda j: (0, j, 0)),
            pl.BlockSpec((2, NB, HD), lambda j: (0, j, 0)),
        ],
        out_shape=[
            jax.ShapeDtypeStruct((4, N, 1), jnp.float32),
            jax.ShapeDtypeStruct((2, N, HD), jnp.float32),
            jax.ShapeDtypeStruct((2, N, HD), jnp.float32),
        ],
    )(x0, degp)


def _hdgfin_body(z0_ref, s_ref, dinv_ref, total_ref, y0_ref):
    dh = dinv_ref[0][None]
    d0 = dinv_ref[1][None]
    tot = 0.5 * (z0_ref[...] + s_ref[...] * dh)
    total_ref[...] = tot
    y0_ref[...] = tot * d0


def _tc_hdgfin(z0, s, dinv):
    return pl.pallas_call(
        _hdgfin_body,
        grid=(GN,),
        in_specs=[
            pl.BlockSpec((2, NB, HD), lambda j: (0, j, 0)),
            pl.BlockSpec((2, NB, HD), lambda j: (0, j, 0)),
            pl.BlockSpec((4, NB, 1), lambda j: (0, j, 0)),
        ],
        out_specs=[
            pl.BlockSpec((2, NB, HD), lambda j: (0, j, 0)),
            pl.BlockSpec((2, NB, HD), lambda j: (0, j, 0)),
        ],
        out_shape=[
            jax.ShapeDtypeStruct((2, N, HD), jnp.float32),
            jax.ShapeDtypeStruct((2, N, HD), jnp.float32),
        ],
    )(z0, s, dinv)


def _make_midscale_body(b):
    def body(s1_ref, dinv_ref, y1_ref):
        db = dinv_ref[1 + b][None]
        y1_ref[...] = s1_ref[...] * (db * db)
    return body


def _tc_midscale(s1, dinv, b):
    return pl.pallas_call(
        _make_midscale_body(b),
        grid=(GN,),
        in_specs=[
            pl.BlockSpec((2, NB, HD), lambda j: (0, j, 0)),
            pl.BlockSpec((4, NB, 1), lambda j: (0, j, 0)),
        ],
        out_specs=pl.BlockSpec((2, NB, HD), lambda j: (0, j, 0)),
        out_shape=jax.ShapeDtypeStruct((2, N, HD), jnp.float32),
    )(s1, dinv)


def _make_combine_body(b, last):
    def body(*refs):
        if b == 0:
            total_ref, s1_ref, s2_ref, dinv_ref, bw_ref = refs[:5]
            outs = refs[5:]
            acc_prev = None
        else:
            total_ref, s1_ref, s2_ref, dinv_ref, bw_ref, acc_ref = refs[:6]
            outs = refs[6:]
            acc_prev = acc_ref[...]
        if last:
            acc_out_ref, = outs
        else:
            total_out_ref, acc_out_ref, ynext_ref = outs

        db = dinv_ref[1 + b][None]
        total = total_ref[...]
        h1 = s1_ref[...] * db
        h2 = s2_ref[...] * db
        layer = (total + h1 + h2) * (1.0 / 3.0)
        ss = jnp.sum(layer * layer, axis=(0, 2))
        scale = (1.0 / jnp.maximum(jnp.sqrt(ss), 1e-12))[None, :, None]
        tot2 = total + layer * scale
        sw = 1.0 / (1.0 + jnp.exp(-bw_ref[b]))
        acc2 = sw * tot2 if acc_prev is None else acc_prev + sw * tot2
        acc_out_ref[...] = acc2
        if not last:
            total_out_ref[...] = tot2
            ynext_ref[...] = tot2 * dinv_ref[2 + b][None]
    return body


def _tc_combine(total, s1, s2, dinv, bw, acc_prev, b):
    last = b == 2
    blk = pl.BlockSpec((2, NB, HD), lambda j: (0, j, 0))
    in_specs = [
        blk, blk, blk,
        pl.BlockSpec((4, NB, 1), lambda j: (0, j, 0)),
        pl.BlockSpec(memory_space=pltpu.SMEM),
    ]
    args = [total, s1, s2, dinv, bw]
    if b > 0:
        in_specs.append(blk)
        args.append(acc_prev)
    if last:
        out_specs = [blk]
        out_shape = [jax.ShapeDtypeStruct((2, N, HD), jnp.float32)]
    else:
        out_specs = [blk, blk, blk]
        out_shape = [jax.ShapeDtypeStruct((2, N, HD), jnp.float32)] * 3
    res = pl.pallas_call(
        _make_combine_body(b, last),
        grid=(GN,),
        in_specs=in_specs,
        out_specs=out_specs,
        out_shape=out_shape,
    )(*args)
    if last:
        return None, res[0], None
    return res[0], res[1], res[2]


BJ = 1024
GJ = B // BJ


def _loss_body(g_ref, p_ref, s1_ref, s2_ref):
    j = pl.program_id(0)
    g = g_ref[...]
    u = g[:, 0]
    i1 = g[:, 1]
    i2 = g[:, 2]
    sp = jnp.sum(u * i1, axis=(0, 2))
    sn = jnp.sum(u * i2, axis=(0, 2))
    z = sp - sn
    vals = jnp.where(z > 0.0, -jnp.log1p(jnp.exp(-z)), z - jnp.log1p(jnp.exp(z)))
    m = jnp.any(p_ref[...] != 0, axis=0).astype(jnp.float32)

    @pl.when(j == 0)
    def _():
        s1_ref[...] = jnp.zeros((1, 1), jnp.float32)
        s2_ref[...] = jnp.zeros((1, 1), jnp.float32)

    s1_ref[...] += jnp.sum(vals * m).reshape(1, 1)
    s2_ref[...] += jnp.sum(m).reshape(1, 1)


def _tc_loss(gath, pairT):
    return pl.pallas_call(
        _loss_body,
        grid=(GJ,),
        in_specs=[
            pl.BlockSpec((2, 3, BJ, HD), lambda j: (0, 0, j, 0)),
            pl.BlockSpec((3, BJ), lambda j: (0, j)),
        ],
        out_specs=[
            pl.BlockSpec((1, 1), lambda j: (0, 0)),
            pl.BlockSpec((1, 1), lambda j: (0, 0)),
        ],
        out_shape=[
            jax.ShapeDtypeStruct((1, 1), jnp.float32),
            jax.ShapeDtypeStruct((1, 1), jnp.float32),
        ],
    )(gath, pairT)


BU = 2048
GU = -(-NU // BU)


def _frob_body(u_ref, i_ref, su_ref, si_ref):
    j = pl.program_id(0)
    rows = lax.broadcasted_iota(jnp.int32, (BU, D), 0) + j * BU
    msk = (rows < NU).astype(jnp.float32)
    u = u_ref[...] * msk
    v = i_ref[...] * msk

    @pl.when(j == 0)
    def _():
        su_ref[...] = jnp.zeros((1, 1), jnp.float32)
        si_ref[...] = jnp.zeros((1, 1), jnp.float32)

    su_ref[...] += jnp.sum(u * u).reshape(1, 1)
    si_ref[...] += jnp.sum(v * v).reshape(1, 1)


def _tc_frob(user_emb, item_emb):
    return pl.pallas_call(
        _frob_body,
        grid=(GU,),
        in_specs=[
            pl.BlockSpec((BU, D), lambda j: (j, 0)),
            pl.BlockSpec((BU, D), lambda j: (j, 0)),
        ],
        out_specs=[
            pl.BlockSpec((1, 1), lambda j: (0, 0)),
            pl.BlockSpec((1, 1), lambda j: (0, 0)),
        ],
        out_shape=[
            jax.ShapeDtypeStruct((1, 1), jnp.float32),
            jax.ShapeDtypeStruct((1, 1), jnp.float32),
        ],
    )(user_emb, item_emb)


def kernel(user_emb, item_emb, behavior_weights, batch_data,
           edges_view, edges_cart, edges_buy):
    f32 = jnp.float32
    _prop1 = _make_prop_kernel(1)
    _prop3 = _make_prop_kernel(3)
    eu = [e[0] for e in (edges_view, edges_cart, edges_buy)]
    ei = [e[1] for e in (edges_view, edges_cart, edges_buy)]
    zeros_deg = jnp.zeros((DROWS,), f32)
    zeros_prop = jnp.zeros((ZR, HD), f32)
    x0 = jnp.concatenate([user_emb, item_emb], axis=0)

    degp = _deg_kernel()(eu[0], ei[0], eu[1], ei[1], eu[2], ei[2], zeros_deg)
    degp = degp.reshape(2, DPAD)[:, :3 * N].reshape(2, 3, N, 1)
    dinv, z0, y = _tc_prep(x0, degp)

    s = _prop3(y.reshape(2 * N, HD), eu[0], ei[0], eu[1], ei[1],
               eu[2], ei[2], zeros_prop)
    total, ynext = _tc_hdgfin(z0, s.reshape(2, NPAD, HD), dinv)

    acc = None
    for b in range(3):
        s1 = _prop1(ynext.reshape(2 * N, HD), eu[b], ei[b], zeros_prop)
        s1 = s1.reshape(2, NPAD, HD)
        y1 = _tc_midscale(s1, dinv, b)
        s2 = _prop1(y1.reshape(2 * N, HD), eu[b], ei[b], zeros_prop)
        s2 = s2.reshape(2, NPAD, HD)
        total, acc, ynext = _tc_combine(total, s1, s2, dinv,
                                        behavior_weights, acc, b)

    pairT = batch_data[:, -1, :3].T
    gath = _gath_kernel()(acc.reshape(2 * N, HD), pairT)
    s1_, s2_ = _tc_loss(gath.reshape(2, 3, B, HD), pairT)
    su, si = _tc_frob(user_emb, item_emb)

    bpr = -s1_[0, 0] / s2_[0, 0]
    emb = (jnp.sqrt(su[0, 0]) + jnp.sqrt(si[0, 0])) / (N_ITEMS + 1)
    return bpr + REG_WEIGHT * emb


# speedup vs baseline: 44.8334x; 1.0483x over previous
import functools

import jax
import jax.numpy as jnp
from jax import lax
from jax.experimental import pallas as pl
from jax.experimental.pallas import tpu as pltpu
from jax.experimental.pallas import tpu_sc as plsc

N_USERS = 25000
N_ITEMS = 25000
D = 64
E = 800000
N_BEH = 3
B = 4096
REG_WEIGHT = 1e-3
NU = N_USERS + 1
N = NU + (N_ITEMS + 1)
HD = D // 2

CH = 128
NCHUNK = E // CH
NSUB = 16
NCORE = 2

NPAD = 51200
TROWS = NPAD // NSUB
ZR = CH
BLK = 256
NJ = BLK // CH
NBLK = E // BLK
DPAD = 150016
DROWS = DPAD // NSUB

@functools.lru_cache(maxsize=None)
def _mesh():
    return plsc.VectorSubcoreMesh(core_axis_name="c", subcore_axis_name="s",
                                  num_cores=NCORE, num_subcores=NSUB)


_SC_PARAMS = pltpu.CompilerParams(use_tc_tiling_on_sc=False,
                                  needs_layout_passes=False)


def _build_idx(dst_ref, src_ref, offset):
    for t in range(CH // 16):
        sl = pl.ds(t * 16, 16)
        dst_ref[sl] = src_ref[sl] + offset


def _deg_body(eu0, ei0, eu1, ei1, eu2, ei2, zeros, out,
              eu_buf, ei_buf, gu2d, gi2d, ones, vbuf, acc, seme, sems):
    c = lax.axis_index("c")
    s = lax.axis_index("s")
    wid = s * NCORE + c

    for t in range(CH // 16):
        ones[pl.ds(t * 16, 16)] = jnp.full((16,), 1.0, jnp.float32)

    lo = s * DROWS
    pltpu.sync_copy(zeros, vbuf)
    pltpu.sync_copy(vbuf, acc.at[pl.ds(lo, DROWS)])

    plsc.subcore_barrier()

    def drain_scatters():
        for j in range(NJ):
            pltpu.make_async_copy(ones, acc.at[gu2d.at[j]], sems).wait()
            pltpu.make_async_copy(ones, acc.at[gi2d.at[j]], sems).wait()

    for b, (eu, ei) in enumerate(((eu0, ei0), (eu1, ei1), (eu2, ei2))):
        bN = b * N
        nk = (NBLK - wid + 31) // 32

        def fire_edges(k1, eu=eu, ei=ei):
            off = (wid + 32 * k1) * BLK
            pltpu.async_copy(eu.at[pl.ds(off, BLK)], eu_buf, seme)
            pltpu.async_copy(ei.at[pl.ds(off, BLK)], ei_buf, seme)

        @pl.when(nk > 0)
        def _():
            fire_edges(0)

        def blk(k, _, eu=eu, ei=ei, bN=bN, fire_edges=fire_edges, nk=nk):
            @pl.when(k > 0)
            def _():
                drain_scatters()

            off = (wid + 32 * k) * BLK
            pltpu.make_async_copy(eu.at[pl.ds(off, BLK)], eu_buf, seme).wait()
            pltpu.make_async_copy(ei.at[pl.ds(off, BLK)], ei_buf, seme).wait()
            for j in range(NJ):
                for t in range(CH // 16):
                    sl = pl.ds(j * CH + t * 16, 16)
                    d16 = pl.ds(t * 16, 16)
                    gu2d[j, d16] = eu_buf[sl] + bN
                    gi2d[j, d16] = ei_buf[sl] + (bN + NU)

            @pl.when(k + 1 < nk)
            def _():
                fire_edges(k + 1)

            for j in range(NJ):
                pltpu.async_copy(ones, acc.at[gu2d.at[j]], sems, add=True)
                pltpu.async_copy(ones, acc.at[gi2d.at[j]], sems, add=True)
            return 0

        lax.fori_loop(0, nk, blk, 0)

        @pl.when(nk > 0)
        def _():
            drain_scatters()

    plsc.subcore_barrier()

    pltpu.sync_copy(acc.at[pl.ds(lo, DROWS)], vbuf)
    pltpu.sync_copy(vbuf, out.at[pl.ds(c * DPAD + lo, DROWS)])


@functools.lru_cache(maxsize=None)
def _deg_kernel():
    return functools.partial(
        pl.kernel,
        out_type=jax.ShapeDtypeStruct((NCORE * DPAD,), jnp.float32),
        mesh=_mesh(),
        compiler_params=_SC_PARAMS,
        scratch_types=[
            pltpu.VMEM((BLK,), jnp.int32),
            pltpu.VMEM((BLK,), jnp.int32),
            pltpu.VMEM((NJ, CH), jnp.int32),
            pltpu.VMEM((NJ, CH), jnp.int32),
            pltpu.VMEM((CH,), jnp.float32),
            pltpu.VMEM((DROWS,), jnp.float32),
            pltpu.VMEM_SHARED((DPAD,), jnp.float32),
            pltpu.SemaphoreType.DMA,
            pltpu.SemaphoreType.DMA,
        ],
    )(_deg_body)


def _make_prop_body(n_edges, toff, scaled):
    def body(*refs):
        table = refs[0]
        pairs = [(refs[1 + 2 * j], refs[2 + 2 * j]) for j in range(n_edges)]
        nin = 1 + 2 * n_edges
        zeros = refs[nin]
        if scaled:
            dsqv = refs[nin + 1]
            out = refs[nin + 2]
            yout = refs[nin + 3]
            (eu_buf, ei_buf, gu2d, su2d, gi2d, si2d, rows_u, rows_i,
             w_buf, acc, semg, sems, seme) = refs[nin + 4:]
        else:
            out = refs[nin + 1]
            (eu_buf, ei_buf, gu2d, su2d, gi2d, si2d, rows_u, rows_i,
             acc, semg, sems, seme) = refs[nin + 2:]

        c = lax.axis_index("c")
        s = lax.axis_index("s")
        cN = c * toff

        lo = s * TROWS
        pltpu.sync_copy(zeros, rows_u.at[pl.ds(0, ZR)])

        def zchunk(k, _):
            pltpu.sync_copy(rows_u.at[pl.ds(0, ZR)],
                            acc.at[pl.ds(lo + k * ZR, ZR)])
            return 0

        lax.fori_loop(0, TROWS // ZR, zchunk, 0)

        plsc.subcore_barrier()

        def drain_scatters():
            for j in range(NJ):
                r = pl.ds(j * CH, CH)
                pltpu.make_async_copy(
                    rows_u.at[r], acc.at[si2d.at[j]], sems).wait()
                pltpu.make_async_copy(
                    rows_i.at[r], acc.at[su2d.at[j]], sems).wait()

        for eu, ei in pairs:
            nk = (NBLK - s + NSUB - 1) // NSUB

            def fire_edges(k, eu=eu, ei=ei):
                off = (s + NSUB * k) * BLK
                pltpu.async_copy(eu.at[pl.ds(off, BLK)], eu_buf, seme)
                pltpu.async_copy(ei.at[pl.ds(off, BLK)], ei_buf, seme)

            @pl.when(nk > 0)
            def _():
                fire_edges(0)

            def blk(k, _, eu=eu, ei=ei):
                @pl.when(k > 0)
                def _():
                    drain_scatters()

                off = (s + NSUB * k) * BLK
                pltpu.make_async_copy(
                    eu.at[pl.ds(off, BLK)], eu_buf, seme).wait()
                pltpu.make_async_copy(
                    ei.at[pl.ds(off, BLK)], ei_buf, seme).wait()
                for j in range(NJ):
                    for t in range(CH // 16):
                        sl = pl.ds(j * CH + t * 16, 16)
                        d16 = pl.ds(t * 16, 16)
                        u = eu_buf[sl]
                        i = ei_buf[sl]
                        gu2d[j, d16] = u + cN
                        su2d[j, d16] = u
                        gi2d[j, d16] = i + (NU + cN)
                        si2d[j, d16] = i + NU

                @pl.when(k + 1 < nk)
                def _():
                    fire_edges_dyn(k + 1)

                gds = []
                for j in range(NJ):
                    r = pl.ds(j * CH, CH)
                    gds.append(pltpu.async_copy(
                        table.at[gu2d.at[j]], rows_u.at[r], semg))
                    gds.append(pltpu.async_copy(
                        table.at[gi2d.at[j]], rows_i.at[r], semg))
                for j in range(NJ):
                    r = pl.ds(j * CH, CH)
                    gds[2 * j].wait()
                    pltpu.async_copy(
                        rows_u.at[r], acc.at[si2d.at[j]], sems, add=True)
                    gds[2 * j + 1].wait()
                    pltpu.async_copy(
                        rows_i.at[r], acc.at[su2d.at[j]], sems, add=True)
                return 0

            def fire_edges_dyn(k1, eu=eu, ei=ei):
                off = (s + NSUB * k1) * BLK
                pltpu.async_copy(eu.at[pl.ds(off, BLK)], eu_buf, seme)
                pltpu.async_copy(ei.at[pl.ds(off, BLK)], ei_buf, seme)

            lax.fori_loop(0, nk, blk, 0)

            @pl.when(nk > 0)
            def _():
                drain_scatters()

        plsc.subcore_barrier()

        def ochunk(k, _):
            row0 = lo + k * ZR
            pltpu.sync_copy(acc.at[pl.ds(row0, ZR)], rows_u.at[pl.ds(0, ZR)])
            pltpu.sync_copy(rows_u.at[pl.ds(0, ZR)],
                            out.at[pl.ds(c * NPAD + row0, ZR)])
            if scaled:
                pltpu.sync_copy(dsqv.at[pl.ds(row0, ZR)], w_buf)
                for r in range(ZR):
                    wv = plsc.load_gather(
                        w_buf, [jnp.full((16,), r, jnp.int32)])
                    for h in range(HD // 16):
                        dh16 = pl.ds(h * 16, 16)
                        rows_i[r, dh16] = rows_u[r, dh16] * wv
                pltpu.sync_copy(rows_i.at[pl.ds(0, ZR)],
                                yout.at[pl.ds(c * NPAD + row0, ZR)])
            return 0

        lax.fori_loop(0, TROWS // ZR, ochunk, 0)

    return body


@functools.lru_cache(maxsize=None)
def _make_prop_kernel(n_edges, toff, scaled=False):
    body = _make_prop_body(n_edges, toff, scaled)
    osh = jax.ShapeDtypeStruct((NCORE * NPAD, HD), jnp.float32)
    scratch = [
        pltpu.VMEM((BLK,), jnp.int32),
        pltpu.VMEM((BLK,), jnp.int32),
        pltpu.VMEM((NJ, CH), jnp.int32),
        pltpu.VMEM((NJ, CH), jnp.int32),
        pltpu.VMEM((NJ, CH), jnp.int32),
        pltpu.VMEM((NJ, CH), jnp.int32),
        pltpu.VMEM((BLK, HD), jnp.float32),
        pltpu.VMEM((BLK, HD), jnp.float32),
    ]
    if scaled:
        scratch.append(pltpu.VMEM((ZR,), jnp.float32))
    scratch += [
        pltpu.VMEM_SHARED((NPAD, HD), jnp.float32),
        pltpu.SemaphoreType.DMA,
        pltpu.SemaphoreType.DMA,
        pltpu.SemaphoreType.DMA,
    ]
    return functools.partial(
        pl.kernel,
        out_type=[osh, osh] if scaled else osh,
        mesh=_mesh(),
        compiler_params=_SC_PARAMS,
        scratch_types=scratch,
    )(body)


def _gath_body(table, pairT, out, jbuf, gbuf, rows, sem):
    c = lax.axis_index("c")
    s = lax.axis_index("s")
    cN = c * N
    per_tile = B // NSUB
    for k in range(3):
        for q in range(per_tile // CH):
            off = s * per_tile + q * CH
            pltpu.sync_copy(pairT.at[k, pl.ds(off, CH)], jbuf)
            _build_idx(gbuf, jbuf, cN if k == 0 else cN + NU)
            pltpu.async_copy(table.at[gbuf], rows, sem).wait()
            base = c * (3 * B) + k * B + off
            pltpu.sync_copy(rows, out.at[pl.ds(base, CH)])


@functools.lru_cache(maxsize=None)
def _gath_kernel():
    return functools.partial(
        pl.kernel,
        out_type=jax.ShapeDtypeStruct((NCORE * 3 * B, HD), jnp.float32),
        mesh=_mesh(),
        compiler_params=_SC_PARAMS,
        scratch_types=[
            pltpu.VMEM((CH,), jnp.int32),
            pltpu.VMEM((CH,), jnp.int32),
            pltpu.VMEM((CH, HD), jnp.float32),
            pltpu.SemaphoreType.DMA,
        ],
    )(_gath_body)


NB = 2048
GN = -(-N // NB)


def _dinv_of(d):
    return jnp.where(d > 0.0, lax.rsqrt(jnp.maximum(d, 1.0)), 0.0)


def _prep_body(x0_ref, degp_ref, dinv_ref, dsq_ref, z0_ref, y_ref,
               su_ref, si_ref):
    j = pl.program_id(0)
    deg = degp_ref[...]
    degs = deg[0] + deg[1]
    degh = degs[0] + degs[1] + degs[2]
    dh = _dinv_of(degh)
    dinv_ref[0] = dh
    dsq_ref[0] = dh * dh
    for b in range(3):
        db = _dinv_of(degs[b])
        dinv_ref[1 + b] = db
        dsq_ref[1 + b] = db * db
    x = x0_ref[...]
    z0_ref[0] = x[:, :HD]
    z0_ref[1] = x[:, HD:]
    y_ref[0] = x[:, :HD] * dh
    y_ref[1] = x[:, HD:] * dh

    rows = lax.broadcasted_iota(jnp.int32, (NB, D), 0) + j * NB
    x2 = x * x

    @pl.when(j == 0)
    def _():
        su_ref[...] = jnp.zeros((1, 1), jnp.float32)
        si_ref[...] = jnp.zeros((1, 1), jnp.float32)

    su_ref[...] += jnp.sum(jnp.where(rows < NU, x2, 0.0)).reshape(1, 1)
    si_ref[...] += jnp.sum(
        jnp.where((rows >= NU) & (rows < N), x2, 0.0)).reshape(1, 1)


def _tc_prep(x0, degp):
    return pl.pallas_call(
        _prep_body,
        grid=(GN,),
        in_specs=[
            pl.BlockSpec((NB, D), lambda j: (j, 0)),
            pl.BlockSpec((2, 3, NB, 1), lambda j: (0, 0, j, 0)),
        ],
        out_specs=[
            pl.BlockSpec((4, NB, 1), lambda j: (0, j, 0)),
            pl.BlockSpec((4, NB, 1), lambda j: (0, j, 0)),
            pl.BlockSpec((2, NB, HD), lambda j: (0, j, 0)),
            pl.BlockSpec((2, NB, HD), lambda j: (0, j, 0)),
            pl.BlockSpec((1, 1), lambda j: (0, 0)),
            pl.BlockSpec((1, 1), lambda j: (0, 0)),
        ],
        out_shape=[
            jax.ShapeDtypeStruct((4, N, 1), jnp.float32),
            jax.ShapeDtypeStruct((4, NPAD, 1), jnp.float32),
            jax.ShapeDtypeStruct((2, N, HD), jnp.float32),
            jax.ShapeDtypeStruct((2, N, HD), jnp.float32),
            jax.ShapeDtypeStruct((1, 1), jnp.float32),
            jax.ShapeDtypeStruct((1, 1), jnp.float32),
        ],
    )(x0, degp)


def _hdgfin_body(z0_ref, s_ref, dinv_ref, total_ref, y0_ref):
    dh = dinv_ref[0][None]
    d0 = dinv_ref[1][None]
    tot = 0.5 * (z0_ref[...] + s_ref[...] * dh)
    total_ref[...] = tot
    y0_ref[...] = tot * d0


def _tc_hdgfin(z0, s, dinv):
    return pl.pallas_call(
        _hdgfin_body,
        grid=(GN,),
        in_specs=[
            pl.BlockSpec((2, NB, HD), lambda j: (0, j, 0)),
            pl.BlockSpec((2, NB, HD), lambda j: (0, j, 0)),
            pl.BlockSpec((4, NB, 1), lambda j: (0, j, 0)),
        ],
        out_specs=[
            pl.BlockSpec((2, NB, HD), lambda j: (0, j, 0)),
            pl.BlockSpec((2, NB, HD), lambda j: (0, j, 0)),
        ],
        out_shape=[
            jax.ShapeDtypeStruct((2, N, HD), jnp.float32),
            jax.ShapeDtypeStruct((2, N, HD), jnp.float32),
        ],
    )(z0, s, dinv)


def _make_combine_body(b, last):
    def body(*refs):
        if b == 0:
            total_ref, s1_ref, s2_ref, dinv_ref, bw_ref = refs[:5]
            outs = refs[5:]
            acc_prev = None
        else:
            total_ref, s1_ref, s2_ref, dinv_ref, bw_ref, acc_ref = refs[:6]
            outs = refs[6:]
            acc_prev = acc_ref[...]
        if last:
            acc_out_ref, = outs
        else:
            total_out_ref, acc_out_ref, ynext_ref = outs

        db = dinv_ref[1 + b][None]
        total = total_ref[...]
        h1 = s1_ref[...] * db
        h2 = s2_ref[...] * db
        layer = (total + h1 + h2) * (1.0 / 3.0)
        ss = jnp.sum(layer * layer, axis=(0, 2))
        scale = (1.0 / jnp.maximum(jnp.sqrt(ss), 1e-12))[None, :, None]
        tot2 = total + layer * scale
        sw = 1.0 / (1.0 + jnp.exp(-bw_ref[b]))
        acc2 = sw * tot2 if acc_prev is None else acc_prev + sw * tot2
        acc_out_ref[...] = acc2
        if not last:
            total_out_ref[...] = tot2
            ynext_ref[...] = tot2 * dinv_ref[2 + b][None]
    return body


def _tc_combine(total, s1, s2, dinv, bw, acc_prev, b):
    last = b == 2
    blk = pl.BlockSpec((2, NB, HD), lambda j: (0, j, 0))
    in_specs = [
        blk, blk, blk,
        pl.BlockSpec((4, NB, 1), lambda j: (0, j, 0)),
        pl.BlockSpec(memory_space=pltpu.SMEM),
    ]
    args = [total, s1, s2, dinv, bw]
    if b > 0:
        in_specs.append(blk)
        args.append(acc_prev)
    if last:
        out_specs = [blk]
        out_shape = [jax.ShapeDtypeStruct((2, N, HD), jnp.float32)]
    else:
        out_specs = [blk, blk, blk]
        out_shape = [jax.ShapeDtypeStruct((2, N, HD), jnp.float32)] * 3
    res = pl.pallas_call(
        _make_combine_body(b, last),
        grid=(GN,),
        in_specs=in_specs,
        out_specs=out_specs,
        out_shape=out_shape,
    )(*args)
    if last:
        return None, res[0], None
    return res[0], res[1], res[2]


BJ = 1024
GJ = B // BJ


def _loss_body(g_ref, p_ref, s1_ref, s2_ref):
    j = pl.program_id(0)
    g = g_ref[...]
    u = g[:, 0]
    i1 = g[:, 1]
    i2 = g[:, 2]
    sp = jnp.sum(u * i1, axis=(0, 2))
    sn = jnp.sum(u * i2, axis=(0, 2))
    z = sp - sn
    vals = jnp.where(z > 0.0, -jnp.log1p(jnp.exp(-z)), z - jnp.log1p(jnp.exp(z)))
    m = jnp.any(p_ref[...] != 0, axis=0).astype(jnp.float32)

    @pl.when(j == 0)
    def _():
        s1_ref[...] = jnp.zeros((1, 1), jnp.float32)
        s2_ref[...] = jnp.zeros((1, 1), jnp.float32)

    s1_ref[...] += jnp.sum(vals * m).reshape(1, 1)
    s2_ref[...] += jnp.sum(m).reshape(1, 1)


def _tc_loss(gath, pairT):
    return pl.pallas_call(
        _loss_body,
        grid=(GJ,),
        in_specs=[
            pl.BlockSpec((2, 3, BJ, HD), lambda j: (0, 0, j, 0)),
            pl.BlockSpec((3, BJ), lambda j: (0, j)),
        ],
        out_specs=[
            pl.BlockSpec((1, 1), lambda j: (0, 0)),
            pl.BlockSpec((1, 1), lambda j: (0, 0)),
        ],
        out_shape=[
            jax.ShapeDtypeStruct((1, 1), jnp.float32),
            jax.ShapeDtypeStruct((1, 1), jnp.float32),
        ],
    )(gath, pairT)


def kernel(user_emb, item_emb, behavior_weights, batch_data,
           edges_view, edges_cart, edges_buy):
    f32 = jnp.float32
    _prop3 = _make_prop_kernel(3, N)
    _prop1s = _make_prop_kernel(1, N, True)
    _prop2 = _make_prop_kernel(1, NPAD)
    eu = [e[0] for e in (edges_view, edges_cart, edges_buy)]
    ei = [e[1] for e in (edges_view, edges_cart, edges_buy)]
    zeros_deg = jnp.zeros((DROWS,), f32)
    zeros_prop = jnp.zeros((ZR, HD), f32)
    x0 = jnp.concatenate([user_emb, item_emb], axis=0)

    degp = _deg_kernel()(eu[0], ei[0], eu[1], ei[1], eu[2], ei[2], zeros_deg)
    degp = degp.reshape(2, DPAD)[:, :3 * N].reshape(2, 3, N, 1)
    dinv, dsq, z0, y, su, si = _tc_prep(x0, degp)

    s = _prop3(y.reshape(2 * N, HD), eu[0], ei[0], eu[1], ei[1],
               eu[2], ei[2], zeros_prop)
    total, ynext = _tc_hdgfin(z0, s.reshape(2, NPAD, HD), dinv)

    acc = None
    for b in range(3):
        s1, y1 = _prop1s(ynext.reshape(2 * N, HD), eu[b], ei[b],
                         zeros_prop, dsq[1 + b, :, 0])
        s2 = _prop2(y1, eu[b], ei[b], zeros_prop)
        total, acc, ynext = _tc_combine(total, s1.reshape(2, NPAD, HD),
                                        s2.reshape(2, NPAD, HD), dinv,
                                        behavior_weights, acc, b)

    pairT = batch_data[:, -1, :3].T
    gath = _gath_kernel()(acc.reshape(2 * N, HD), pairT)
    s1_, s2_ = _tc_loss(gath.reshape(2, 3, B, HD), pairT)

    bpr = -s1_[0, 0] / s2_[0, 0]
    emb = (jnp.sqrt(su[0, 0]) + jnp.sqrt(si[0, 0])) / (N_ITEMS + 1)
    return bpr + REG_WEIGHT * emb
